# Initial kernel scaffold; baseline (speedup 1.0000x reference)
import jax
import jax.numpy as jnp
from jax.experimental import pallas as pl


def kernel(x, x_mask, x_idx, keys_store, store_vals, neighbor_feats, W_enc, b_enc, W_cls, b_cls, W1, b1, W2, b2):
    raise NotImplementedError("write your pallas kernel here")


# speedup vs baseline: 5.1762x; 5.1762x over previous
import functools

import jax
import jax.numpy as jnp
from jax.experimental import pallas as pl

B = 128
S = 128
D = 768
K_STORE = 100000
NUM_CLASSES = 1000
K = 32
TEMP = 10.0

TILE_N = 2048
N_PAD = 100352
N_TILES = N_PAD // TILE_N


def _score_body(q_ref, k_ref, xidx_ref, s_ref):
    j = pl.program_id(0)
    q = q_ref[...]
    kb = k_ref[...]
    dot = jax.lax.dot_general(q, kb, (((1,), (1,)), ((), ())),
                              preferred_element_type=jnp.float32)
    k2 = jnp.sum(kb * kb, axis=1)
    s = 2.0 * dot - k2[None, :]
    col = jax.lax.broadcasted_iota(jnp.int32, (B, TILE_N), 1) + j * TILE_N
    self_col = xidx_ref[...]
    mask = (col == self_col) | (col >= K_STORE)
    s_ref[...] = jnp.where(mask, -3.0e38, s)


@jax.jit
def _scores(q, keys_pad, x_idx):
    return pl.pallas_call(
        _score_body,
        grid=(N_TILES,),
        in_specs=[
            pl.BlockSpec((B, D), lambda j: (0, 0)),
            pl.BlockSpec((TILE_N, D), lambda j: (j, 0)),
            pl.BlockSpec((B, 1), lambda j: (0, 0)),
        ],
        out_specs=pl.BlockSpec((B, TILE_N), lambda j: (0, j)),
        out_shape=jax.ShapeDtypeStruct((B, N_PAD), jnp.float32),
    )(q, keys_pad, x_idx.astype(jnp.int32).reshape(B, 1))


def kernel(x, x_mask, x_idx, keys_store, store_vals, neighbor_feats,
           W_enc, b_enc, W_cls, b_cls, W1, b1, W2, b2):
    m = x_mask[:, :, None]
    pooled = jnp.sum(x * m, axis=1) / jnp.maximum(jnp.sum(m, axis=1), 1.0)
    text_rep = jnp.tanh(pooled @ W_enc + b_enc)
    model_prob = jax.nn.softmax(text_rep @ W_cls + b_cls, axis=-1)

    q = keys_store[x_idx]
    keys_pad = jnp.pad(keys_store, ((0, N_PAD - K_STORE), (0, 0)))
    scores = _scores(q, keys_pad, x_idx)

    _, knns = jax.lax.top_k(scores, K)

    knn_keys = keys_store[knns]
    dists = jnp.sum((text_rep[:, None, :] - knn_keys) ** 2, axis=-1)
    probs = jax.nn.softmax(-dists / TEMP, axis=-1)
    labels = store_vals[knns]
    rows = jnp.broadcast_to(jnp.arange(B)[:, None], (B, K))
    knn_prob = jnp.zeros((B, NUM_CLASSES), dtype=probs.dtype).at[rows, labels].add(probs)

    neighbors = jnp.tanh(neighbor_feats[knns] @ W_enc + b_enc)
    neighbor_rep = jnp.sum(probs[:, :, None] * neighbors, axis=1)

    h = jnp.concatenate([text_rep, neighbor_rep], axis=-1)
    h = h @ W1 + b1
    p_knn = jax.nn.sigmoid(h @ W2 + b2)

    return jnp.log(p_knn * knn_prob + (1.0 - p_knn) * model_prob + 1e-12)


# speedup vs baseline: 13.8220x; 2.6703x over previous
import functools

import jax
import jax.numpy as jnp
from jax import lax
from jax.experimental import pallas as pl
from jax.experimental.pallas import tpu as pltpu, tpu_sc as plsc

B = 128
S = 128
D = 768
K_STORE = 100000
NUM_CLASSES = 1000
K = 32
TEMP = 10.0

TILE_N = 2048
N_PAD = 100352
N_TILES = N_PAD // TILE_N
G = N_PAD // 128
NCHUNK = G // 16

_NEG = -3.0e38


def _score_body(q_ref, k_ref, xidx_ref, s_ref, gm_ref):
    j = pl.program_id(0)
    q = q_ref[...]
    kb = k_ref[...]
    dot = jax.lax.dot_general(q, kb, (((1,), (1,)), ((), ())),
                              preferred_element_type=jnp.float32)
    k2 = jnp.sum(kb * kb, axis=1)
    s = 2.0 * dot - k2[None, :]
    col = jax.lax.broadcasted_iota(jnp.int32, (B, TILE_N), 1) + j * TILE_N
    self_col = xidx_ref[...]
    mask = (col == self_col) | (col >= K_STORE)
    s = jnp.where(mask, _NEG, s)
    s_ref[...] = s
    gm_ref[...] = jnp.max(s.reshape(B, TILE_N // 128, 128), axis=2).transpose(1, 0)[None]


@jax.jit
def _scores(q, keys_pad, x_idx):
    return pl.pallas_call(
        _score_body,
        grid=(N_TILES,),
        in_specs=[
            pl.BlockSpec((B, D), lambda j: (0, 0)),
            pl.BlockSpec((TILE_N, D), lambda j: (j, 0)),
            pl.BlockSpec((B, 1), lambda j: (0, 0)),
        ],
        out_specs=[
            pl.BlockSpec((B, TILE_N), lambda j: (0, j)),
            pl.BlockSpec((1, TILE_N // 128, B), lambda j: (j, 0, 0)),
        ],
        out_shape=[
            jax.ShapeDtypeStruct((B, N_PAD), jnp.float32),
            jax.ShapeDtypeStruct((N_TILES, TILE_N // 128, B), jnp.float32),
        ],
    )(q, keys_pad, x_idx.astype(jnp.int32).reshape(B, 1))


def _merge32(ka, va, kb, vb):
    rb0 = lax.rev(kb[0], (0,))
    rb1 = lax.rev(kb[1], (0,))
    rv0 = lax.rev(vb[0], (0,))
    rv1 = lax.rev(vb[1], (0,))
    ge0 = ka[0] >= rb1
    k0 = jnp.where(ge0, ka[0], rb1)
    v0 = jnp.where(ge0, va[0], rv1)
    ge1 = ka[1] >= rb0
    k1 = jnp.where(ge1, ka[1], rb0)
    v1 = jnp.where(ge1, va[1], rv0)
    geh = k0 >= k1
    hk = jnp.where(geh, k0, k1)
    hv = jnp.where(geh, v0, v1)
    lk = jnp.where(geh, k1, k0)
    lv = jnp.where(geh, v1, v0)
    hk, hv = plsc.sort_key_val(hk, hv, descending=True)
    lk, lv = plsc.sort_key_val(lk, lv, descending=True)
    return (hk, lk), (hv, lv)


def _fold_chunk(acc, k16, v16):
    neg = jnp.full((16,), _NEG, jnp.float32)
    zero = jnp.zeros((16,), jnp.int32)
    sk, sv = plsc.sort_key_val(k16, v16, descending=True)
    if acc is None:
        return (sk, neg), (sv, zero)
    ka, va = acc
    return _merge32(ka, va, (sk, neg), (sv, zero))


def _select_body(gm_hbm, scores_hbm, knns_hbm,
                 gm_v, cand_v, idx_v, gids_v, knns_v, sem):
    nc = 2
    wid = lax.axis_index("s") * nc + lax.axis_index("c")
    iota = lax.iota(jnp.int32, 16)

    def row_body(t, carry):
        r = wid * 4 + t
        pltpu.sync_copy(gm_hbm.at[r], gm_v)

        acc = None
        for j in range(NCHUNK):
            k16 = gm_v[pl.ds(16 * j, 16)]
            v16 = iota + 16 * j
            acc = _fold_chunk(acc, k16, v16)
        gk, gv = acc
        gids_v[pl.ds(0, 16)] = gv[0]
        gids_v[pl.ds(16, 16)] = gv[1]
        idx_v[pl.ds(0, 16)] = gv[0] + r * G
        idx_v[pl.ds(16, 16)] = gv[1] + r * G

        pltpu.async_copy(scores_hbm.at[idx_v], cand_v, sem).wait()

        submax = []
        for g in range(32):
            m = cand_v[g, pl.ds(0, 16)]
            for i in range(1, 8):
                m = jnp.maximum(m, cand_v[g, pl.ds(16 * i, 16)])
            submax.append(m)

        acc = None
        for g in range(32):
            acc = _fold_chunk(acc, submax[g], iota + 16 * g)
        pk, pv = acc

        acc = None
        for half in range(2):
            p = pv[half]
            g = lax.shift_right_logical(p, 4)
            j = jnp.bitwise_and(p, 15)
            gidv = plsc.load_gather(gids_v, [g])
            base = gidv * 128 + j
            for i in range(8):
                k16 = plsc.load_gather(cand_v, [g, j + 16 * i])
                acc = _fold_chunk(acc, k16, base + 16 * i)
        fk, fv = acc
        knns_v[pl.ds(0, 16)] = fv[0]
        knns_v[pl.ds(16, 16)] = fv[1]
        pltpu.sync_copy(knns_v, knns_hbm.at[r])
        return carry

    lax.fori_loop(0, 4, row_body, 0)


@jax.jit
def _select(gm1, scores_resh):
    mesh = plsc.VectorSubcoreMesh(core_axis_name="c", subcore_axis_name="s")
    return pl.kernel(
        _select_body,
        mesh=mesh,
        out_type=jax.ShapeDtypeStruct((B, K), jnp.int32),
        scratch_types=[
            pltpu.VMEM((G,), jnp.float32),
            pltpu.VMEM((32, 128), jnp.float32),
            pltpu.VMEM((32,), jnp.int32),
            pltpu.VMEM((32,), jnp.int32),
            pltpu.VMEM((32,), jnp.int32),
            pltpu.SemaphoreType.DMA,
        ],
        compiler_params=pltpu.CompilerParams(needs_layout_passes=False),
    )(gm1, scores_resh)


def kernel(x, x_mask, x_idx, keys_store, store_vals, neighbor_feats,
           W_enc, b_enc, W_cls, b_cls, W1, b1, W2, b2):
    m = x_mask[:, :, None]
    pooled = jnp.sum(x * m, axis=1) / jnp.maximum(jnp.sum(m, axis=1), 1.0)
    text_rep = jnp.tanh(pooled @ W_enc + b_enc)
    model_prob = jax.nn.softmax(text_rep @ W_cls + b_cls, axis=-1)

    q = keys_store[x_idx]
    keys_pad = jnp.pad(keys_store, ((0, N_PAD - K_STORE), (0, 0)))
    scores, gm_t = _scores(q, keys_pad, x_idx)
    gm1 = gm_t.transpose(2, 0, 1).reshape(B, G)

    knns = _select(gm1, scores.reshape(B * G, 128))

    knn_keys = keys_store[knns]
    dists = jnp.sum((text_rep[:, None, :] - knn_keys) ** 2, axis=-1)
    probs = jax.nn.softmax(-dists / TEMP, axis=-1)
    labels = store_vals[knns]
    rows = jnp.broadcast_to(jnp.arange(B)[:, None], (B, K))
    knn_prob = jnp.zeros((B, NUM_CLASSES), dtype=probs.dtype).at[rows, labels].add(probs)

    neighbors = jnp.tanh(neighbor_feats[knns] @ W_enc + b_enc)
    neighbor_rep = jnp.sum(probs[:, :, None] * neighbors, axis=1)

    h = jnp.concatenate([text_rep, neighbor_rep], axis=-1)
    h = h @ W1 + b1
    p_knn = jax.nn.sigmoid(h @ W2 + b2)

    return jnp.log(p_knn * knn_prob + (1.0 - p_knn) * model_prob + 1e-12)


# speedup vs baseline: 15.7692x; 1.1409x over previous
import functools

import jax
import jax.numpy as jnp
from jax import lax
from jax.experimental import pallas as pl
from jax.experimental.pallas import tpu as pltpu, tpu_sc as plsc

B = 128
S = 128
D = 768
K_STORE = 100000
NUM_CLASSES = 1000
K = 32
TEMP = 10.0

TILE_N = 2048
N_PAD = 100352
N_TILES = N_PAD // TILE_N
G = N_PAD // 128
NCHUNK = G // 16

_NEG = -3.0e38


def _score_body(q_ref, k_ref, xidx_ref, s_ref, gm_ref):
    j = pl.program_id(0)
    q = q_ref[...]
    kb = k_ref[...]
    dot = jax.lax.dot_general(q, kb, (((1,), (1,)), ((), ())),
                              preferred_element_type=jnp.float32)
    k2 = jnp.sum(kb * kb, axis=1)
    s = 2.0 * dot - k2[None, :]
    col = jax.lax.broadcasted_iota(jnp.int32, (B, TILE_N), 1) + j * TILE_N
    self_col = xidx_ref[...]
    mask = (col == self_col) | (col >= K_STORE)
    s = jnp.where(mask, _NEG, s)
    s_ref[...] = s
    gm_ref[...] = jnp.max(s.reshape(B, TILE_N // 128, 128), axis=2).transpose(1, 0)[None]


@jax.jit
def _scores(q, keys_pad, x_idx):
    return pl.pallas_call(
        _score_body,
        grid=(N_TILES,),
        in_specs=[
            pl.BlockSpec((B, D), lambda j: (0, 0)),
            pl.BlockSpec((TILE_N, D), lambda j: (j, 0)),
            pl.BlockSpec((B, 1), lambda j: (0, 0)),
        ],
        out_specs=[
            pl.BlockSpec((B, TILE_N), lambda j: (0, j)),
            pl.BlockSpec((1, TILE_N // 128, B), lambda j: (j, 0, 0)),
        ],
        out_shape=[
            jax.ShapeDtypeStruct((B, N_PAD), jnp.float32),
            jax.ShapeDtypeStruct((N_TILES, TILE_N // 128, B), jnp.float32),
        ],
    )(q, keys_pad, x_idx.astype(jnp.int32).reshape(B, 1))


def _merge32(ka, va, kb, vb):
    rb0 = lax.rev(kb[0], (0,))
    rb1 = lax.rev(kb[1], (0,))
    rv0 = lax.rev(vb[0], (0,))
    rv1 = lax.rev(vb[1], (0,))
    ge0 = ka[0] >= rb1
    k0 = jnp.where(ge0, ka[0], rb1)
    v0 = jnp.where(ge0, va[0], rv1)
    ge1 = ka[1] >= rb0
    k1 = jnp.where(ge1, ka[1], rb0)
    v1 = jnp.where(ge1, va[1], rv0)
    geh = k0 >= k1
    hk = jnp.where(geh, k0, k1)
    hv = jnp.where(geh, v0, v1)
    lk = jnp.where(geh, k1, k0)
    lv = jnp.where(geh, v1, v0)
    hk, hv = plsc.sort_key_val(hk, hv, descending=True)
    lk, lv = plsc.sort_key_val(lk, lv, descending=True)
    return (hk, lk), (hv, lv)


def _fold_chunk(acc, k16, v16):
    neg = jnp.full((16,), _NEG, jnp.float32)
    zero = jnp.zeros((16,), jnp.int32)
    sk, sv = plsc.sort_key_val(k16, v16, descending=True)
    if acc is None:
        return (sk, neg), (sv, zero)
    ka, va = acc
    return _merge32(ka, va, (sk, neg), (sv, zero))


def _select_body(gm_hbm, scores_hbm, keys_hbm, nf_hbm, sv_hbm, trep_hbm,
                 probs_hbm, labels_hbm, nfg_hbm,
                 gm_v, cand_v, idx_v, gids_v, knns_v, t_v, krows_v, nfrows_v,
                 labels_v, probs_v, sem):
    nc = 2
    wid = lax.axis_index("s") * nc + lax.axis_index("c")
    iota = lax.iota(jnp.int32, 16)

    def row_body(t, carry):
        r = wid * 4 + t
        pltpu.sync_copy(gm_hbm.at[r], gm_v)

        acc = None
        for j in range(NCHUNK):
            k16 = gm_v[pl.ds(16 * j, 16)]
            v16 = iota + 16 * j
            acc = _fold_chunk(acc, k16, v16)
        gk, gv = acc
        gids_v[pl.ds(0, 16)] = gv[0]
        gids_v[pl.ds(16, 16)] = gv[1]
        idx_v[pl.ds(0, 16)] = gv[0] + r * G
        idx_v[pl.ds(16, 16)] = gv[1] + r * G

        pltpu.async_copy(scores_hbm.at[idx_v], cand_v, sem).wait()

        submax = []
        for g in range(32):
            m = cand_v[g, pl.ds(0, 16)]
            for i in range(1, 8):
                m = jnp.maximum(m, cand_v[g, pl.ds(16 * i, 16)])
            submax.append(m)

        acc = None
        for g in range(32):
            acc = _fold_chunk(acc, submax[g], iota + 16 * g)
        pk, pv = acc

        acc = None
        for half in range(2):
            p = pv[half]
            g = lax.shift_right_logical(p, 4)
            j = jnp.bitwise_and(p, 15)
            gidv = plsc.load_gather(gids_v, [g])
            base = gidv * 128 + j
            for i in range(8):
                k16 = plsc.load_gather(cand_v, [g, j + 16 * i])
                acc = _fold_chunk(acc, k16, base + 16 * i)
        fk, fv = acc
        knns_v[pl.ds(0, 16)] = fv[0]
        knns_v[pl.ds(16, 16)] = fv[1]

        pltpu.sync_copy(trep_hbm.at[r], t_v)
        pltpu.async_copy(keys_hbm.at[knns_v], krows_v, sem).wait()
        pltpu.async_copy(nf_hbm.at[knns_v], nfrows_v, sem).wait()
        pltpu.async_copy(sv_hbm.at[knns_v], labels_v, sem).wait()
        pltpu.sync_copy(nfrows_v, nfg_hbm.at[r])
        pltpu.sync_copy(labels_v, labels_hbm.at[r])

        def dist_body(n, dcarry):
            d0, d1 = dcarry
            acc = jnp.zeros((16,), jnp.float32)
            for i in range(D // 16):
                diff = t_v[pl.ds(16 * i, 16)] - krows_v[n, pl.ds(16 * i, 16)]
                acc = acc + diff * diff
            dn = jnp.sum(acc)
            lane = jnp.bitwise_and(n, 15)
            dsplat = jnp.full((16,), dn, jnp.float32)
            d0 = jnp.where((iota == lane) & (n < 16), dsplat, d0)
            d1 = jnp.where((iota == lane) & (n >= 16), dsplat, d1)
            return d0, d1

        zero16 = jnp.zeros((16,), jnp.float32)
        d0, d1 = lax.fori_loop(0, K, dist_body, (zero16, zero16))

        u0 = jnp.negative(d0) / TEMP
        u1 = jnp.negative(d1) / TEMP
        m = jnp.maximum(jnp.max(u0), jnp.max(u1))
        msp = jnp.full((16,), m, jnp.float32)
        e0 = jnp.exp(u0 - msp)
        e1 = jnp.exp(u1 - msp)
        ssum = jnp.sum(e0) + jnp.sum(e1)
        ssp = jnp.full((16,), ssum, jnp.float32)
        probs_v[pl.ds(0, 16)] = e0 / ssp
        probs_v[pl.ds(16, 16)] = e1 / ssp
        pltpu.sync_copy(probs_v, probs_hbm.at[r])
        return carry

    lax.fori_loop(0, 4, row_body, 0)


@jax.jit
def _select(gm1, scores_resh, keys_store, neighbor_feats, store_vals, text_rep):
    mesh = plsc.VectorSubcoreMesh(core_axis_name="c", subcore_axis_name="s")
    return pl.kernel(
        _select_body,
        mesh=mesh,
        out_type=[
            jax.ShapeDtypeStruct((B, K), jnp.float32),
            jax.ShapeDtypeStruct((B, K), jnp.int32),
            jax.ShapeDtypeStruct((B, K, D), jnp.float32),
        ],
        scratch_types=[
            pltpu.VMEM((G,), jnp.float32),
            pltpu.VMEM((32, 128), jnp.float32),
            pltpu.VMEM((32,), jnp.int32),
            pltpu.VMEM((32,), jnp.int32),
            pltpu.VMEM((32,), jnp.int32),
            pltpu.VMEM((D,), jnp.float32),
            pltpu.VMEM((K, D), jnp.float32),
            pltpu.VMEM((K, D), jnp.float32),
            pltpu.VMEM((K,), jnp.int32),
            pltpu.VMEM((K,), jnp.float32),
            pltpu.SemaphoreType.DMA,
        ],
        compiler_params=pltpu.CompilerParams(needs_layout_passes=False),
    )(gm1, scores_resh, keys_store, neighbor_feats, store_vals, text_rep)


def kernel(x, x_mask, x_idx, keys_store, store_vals, neighbor_feats,
           W_enc, b_enc, W_cls, b_cls, W1, b1, W2, b2):
    m = x_mask[:, :, None]
    pooled = jnp.sum(x * m, axis=1) / jnp.maximum(jnp.sum(m, axis=1), 1.0)
    text_rep = jnp.tanh(pooled @ W_enc + b_enc)
    model_prob = jax.nn.softmax(text_rep @ W_cls + b_cls, axis=-1)

    q = keys_store[x_idx]
    keys_pad = jnp.pad(keys_store, ((0, N_PAD - K_STORE), (0, 0)))
    scores, gm_t = _scores(q, keys_pad, x_idx)
    gm1 = gm_t.transpose(2, 0, 1).reshape(B, G)

    probs, labels, nf_g = _select(gm1, scores.reshape(B * G, 128),
                                  keys_store, neighbor_feats, store_vals,
                                  text_rep)

    rows = jnp.broadcast_to(jnp.arange(B)[:, None], (B, K))
    knn_prob = jnp.zeros((B, NUM_CLASSES), dtype=probs.dtype).at[rows, labels].add(probs)

    neighbors = jnp.tanh(nf_g @ W_enc + b_enc)
    neighbor_rep = jnp.sum(probs[:, :, None] * neighbors, axis=1)

    h = jnp.concatenate([text_rep, neighbor_rep], axis=-1)
    h = h @ W1 + b1
    p_knn = jax.nn.sigmoid(h @ W2 + b2)

    return jnp.log(p_knn * knn_prob + (1.0 - p_knn) * model_prob + 1e-12)


# speedup vs baseline: 28.4608x; 1.8048x over previous
import functools

import jax
import jax.numpy as jnp
from jax import lax
from jax.experimental import pallas as pl
from jax.experimental.pallas import tpu as pltpu, tpu_sc as plsc

B = 128
S = 128
D = 768
K_STORE = 100000
NUM_CLASSES = 1000
K = 32
TEMP = 10.0

TILE_N = 2048
N_PAD = 100352
N_TILES = N_PAD // TILE_N
G = N_PAD // 128
NCHUNK = G // 16

_NEG = -3.0e38


def _score_body(q_ref, k_ref, xidx_ref, s_ref, gm_ref):
    j = pl.program_id(0)
    q = q_ref[...]
    kb = k_ref[...]
    dot = jax.lax.dot_general(q, kb, (((1,), (1,)), ((), ())),
                              preferred_element_type=jnp.float32)
    k2 = jnp.sum(kb * kb, axis=1)
    s = 2.0 * dot - k2[None, :]
    col = jax.lax.broadcasted_iota(jnp.int32, (B, TILE_N), 1) + j * TILE_N
    self_col = xidx_ref[...]
    mask = (col == self_col) | (col >= K_STORE)
    s = jnp.where(mask, _NEG, s)
    s_ref[...] = s
    gm_ref[...] = jnp.max(s.reshape(B, TILE_N // 128, 128), axis=2).transpose(1, 0)[None]


@jax.jit
def _scores(q, keys_pad, x_idx):
    return pl.pallas_call(
        _score_body,
        grid=(N_TILES,),
        in_specs=[
            pl.BlockSpec((B, D), lambda j: (0, 0)),
            pl.BlockSpec((TILE_N, D), lambda j: (j, 0)),
            pl.BlockSpec((B, 1), lambda j: (0, 0)),
        ],
        out_specs=[
            pl.BlockSpec((B, TILE_N), lambda j: (0, j)),
            pl.BlockSpec((1, TILE_N // 128, B), lambda j: (j, 0, 0)),
        ],
        out_shape=[
            jax.ShapeDtypeStruct((B, N_PAD), jnp.float32),
            jax.ShapeDtypeStruct((N_TILES, TILE_N // 128, B), jnp.float32),
        ],
    )(q, keys_pad, x_idx.astype(jnp.int32).reshape(B, 1))


def _merge32(ka, va, kb, vb):
    rb0 = lax.rev(kb[0], (0,))
    rb1 = lax.rev(kb[1], (0,))
    rv0 = lax.rev(vb[0], (0,))
    rv1 = lax.rev(vb[1], (0,))
    ge0 = ka[0] >= rb1
    k0 = jnp.where(ge0, ka[0], rb1)
    v0 = jnp.where(ge0, va[0], rv1)
    ge1 = ka[1] >= rb0
    k1 = jnp.where(ge1, ka[1], rb0)
    v1 = jnp.where(ge1, va[1], rv0)
    geh = k0 >= k1
    hk = jnp.where(geh, k0, k1)
    hv = jnp.where(geh, v0, v1)
    lk = jnp.where(geh, k1, k0)
    lv = jnp.where(geh, v1, v0)
    hk, hv = plsc.sort_key_val(hk, hv, descending=True)
    lk, lv = plsc.sort_key_val(lk, lv, descending=True)
    return (hk, lk), (hv, lv)


def _fold_chunk(acc, k16, v16):
    neg = jnp.full((16,), _NEG, jnp.float32)
    zero = jnp.zeros((16,), jnp.int32)
    sk, sv = plsc.sort_key_val(k16, v16, descending=True)
    if acc is None:
        return (sk, neg), (sv, zero)
    ka, va = acc
    return _merge32(ka, va, (sk, neg), (sv, zero))


def _select_body(gm_hbm, scores_hbm, keys_hbm, nf_hbm, sv_hbm, trep_hbm,
                 probs_hbm, labels_hbm, nfg_hbm,
                 gm_v, cand_v, idx_v, gids_v, knns_v, t_v, krows_v, nfrows_v,
                 labels_v, probs_v, sem):
    nc = 2
    wid = lax.axis_index("s") * nc + lax.axis_index("c")
    iota = lax.iota(jnp.int32, 16)

    def row_body(t, carry):
        r = wid * 4 + t
        pltpu.sync_copy(gm_hbm.at[r], gm_v)

        acc = None
        for j in range(NCHUNK):
            k16 = gm_v[pl.ds(16 * j, 16)]
            v16 = iota + 16 * j
            acc = _fold_chunk(acc, k16, v16)
        gk, gv = acc
        gids_v[pl.ds(0, 16)] = gv[0]
        gids_v[pl.ds(16, 16)] = gv[1]
        idx_v[pl.ds(0, 16)] = gv[0] + r * G
        idx_v[pl.ds(16, 16)] = gv[1] + r * G

        pltpu.async_copy(scores_hbm.at[idx_v], cand_v, sem).wait()

        submax = []
        for g in range(32):
            m = cand_v[g, pl.ds(0, 16)]
            for i in range(1, 8):
                m = jnp.maximum(m, cand_v[g, pl.ds(16 * i, 16)])
            submax.append(m)

        acc = None
        for g in range(32):
            acc = _fold_chunk(acc, submax[g], iota + 16 * g)
        pk, pv = acc

        acc = None
        for half in range(2):
            p = pv[half]
            g = lax.shift_right_logical(p, 4)
            j = jnp.bitwise_and(p, 15)
            gidv = plsc.load_gather(gids_v, [g])
            base = gidv * 128 + j
            for i in range(8):
                k16 = plsc.load_gather(cand_v, [g, j + 16 * i])
                acc = _fold_chunk(acc, k16, base + 16 * i)
        fk, fv = acc
        knns_v[pl.ds(0, 16)] = fv[0]
        knns_v[pl.ds(16, 16)] = fv[1]

        pltpu.sync_copy(trep_hbm.at[r], t_v)
        pltpu.async_copy(keys_hbm.at[knns_v], krows_v, sem).wait()
        pltpu.async_copy(nf_hbm.at[knns_v], nfrows_v, sem).wait()
        pltpu.async_copy(sv_hbm.at[knns_v], labels_v, sem).wait()
        pltpu.sync_copy(nfrows_v, nfg_hbm.at[r])
        pltpu.sync_copy(labels_v, labels_hbm.at[r])

        def dist_body(n, dcarry):
            d0, d1 = dcarry
            acc = jnp.zeros((16,), jnp.float32)
            for i in range(D // 16):
                diff = t_v[pl.ds(16 * i, 16)] - krows_v[n, pl.ds(16 * i, 16)]
                acc = acc + diff * diff
            dn = jnp.sum(acc)
            lane = jnp.bitwise_and(n, 15)
            dsplat = jnp.full((16,), dn, jnp.float32)
            d0 = jnp.where((iota == lane) & (n < 16), dsplat, d0)
            d1 = jnp.where((iota == lane) & (n >= 16), dsplat, d1)
            return d0, d1

        zero16 = jnp.zeros((16,), jnp.float32)
        d0, d1 = lax.fori_loop(0, K, dist_body, (zero16, zero16))

        u0 = jnp.negative(d0) / TEMP
        u1 = jnp.negative(d1) / TEMP
        m = jnp.maximum(jnp.max(u0), jnp.max(u1))
        msp = jnp.full((16,), m, jnp.float32)
        e0 = jnp.exp(u0 - msp)
        e1 = jnp.exp(u1 - msp)
        ssum = jnp.sum(e0) + jnp.sum(e1)
        ssp = jnp.full((16,), ssum, jnp.float32)
        probs_v[pl.ds(0, 16)] = e0 / ssp
        probs_v[pl.ds(16, 16)] = e1 / ssp
        pltpu.sync_copy(probs_v, probs_hbm.at[r])
        return carry

    lax.fori_loop(0, 4, row_body, 0)


@jax.jit
def _select(gm1, scores_resh, keys_store, neighbor_feats, store_vals, text_rep):
    mesh = plsc.VectorSubcoreMesh(core_axis_name="c", subcore_axis_name="s")
    return pl.kernel(
        _select_body,
        mesh=mesh,
        out_type=[
            jax.ShapeDtypeStruct((B, K), jnp.float32),
            jax.ShapeDtypeStruct((B, K), jnp.int32),
            jax.ShapeDtypeStruct((B, K, D), jnp.float32),
        ],
        scratch_types=[
            pltpu.VMEM((G,), jnp.float32),
            pltpu.VMEM((32, 128), jnp.float32),
            pltpu.VMEM((32,), jnp.int32),
            pltpu.VMEM((32,), jnp.int32),
            pltpu.VMEM((32,), jnp.int32),
            pltpu.VMEM((D,), jnp.float32),
            pltpu.VMEM((K, D), jnp.float32),
            pltpu.VMEM((K, D), jnp.float32),
            pltpu.VMEM((K,), jnp.int32),
            pltpu.VMEM((K,), jnp.float32),
            pltpu.SemaphoreType.DMA,
        ],
        compiler_params=pltpu.CompilerParams(needs_layout_passes=False),
    )(gm1, scores_resh, keys_store, neighbor_feats, store_vals, text_rep)


ENC_ROWS = 32


def _encode_body(x_ref, m_ref, wenc_ref, benc_ref, wcls_ref, bcls_ref,
                 t_ref, p_ref):
    xm = x_ref[...] * m_ref[...][:, :, None]
    pooled = jnp.sum(xm, axis=1) / jnp.maximum(
        jnp.sum(m_ref[...], axis=1), 1.0)[:, None]
    t = jnp.tanh(jax.lax.dot_general(pooled, wenc_ref[...],
                                     (((1,), (0,)), ((), ())),
                                     preferred_element_type=jnp.float32)
                 + benc_ref[...])
    logits = jax.lax.dot_general(t, wcls_ref[...], (((1,), (0,)), ((), ())),
                                 preferred_element_type=jnp.float32) + bcls_ref[...]
    mx = jnp.max(logits, axis=1, keepdims=True)
    e = jnp.exp(logits - mx)
    t_ref[...] = t
    p_ref[...] = e / jnp.sum(e, axis=1, keepdims=True)


@jax.jit
def _encode(x, x_mask, W_enc, b_enc, W_cls, b_cls):
    return pl.pallas_call(
        _encode_body,
        grid=(B // ENC_ROWS,),
        in_specs=[
            pl.BlockSpec((ENC_ROWS, S, D), lambda i: (i, 0, 0)),
            pl.BlockSpec((ENC_ROWS, S), lambda i: (i, 0)),
            pl.BlockSpec((D, D), lambda i: (0, 0)),
            pl.BlockSpec((1, D), lambda i: (0, 0)),
            pl.BlockSpec((D, NUM_CLASSES), lambda i: (0, 0)),
            pl.BlockSpec((1, NUM_CLASSES), lambda i: (0, 0)),
        ],
        out_specs=[
            pl.BlockSpec((ENC_ROWS, D), lambda i: (i, 0)),
            pl.BlockSpec((ENC_ROWS, NUM_CLASSES), lambda i: (i, 0)),
        ],
        out_shape=[
            jax.ShapeDtypeStruct((B, D), jnp.float32),
            jax.ShapeDtypeStruct((B, NUM_CLASSES), jnp.float32),
        ],
    )(x, x_mask, W_enc, b_enc.reshape(1, D), W_cls, b_cls.reshape(1, NUM_CLASSES))


TAIL_ROWS = 32


def _tail_body(t_ref, pr_ref, lab_ref, nfg_ref, mp_ref,
               wenc_ref, benc_ref, w1_ref, b1_ref, w2_ref, b2_ref, out_ref):
    nf = nfg_ref[...].reshape(TAIL_ROWS * K, D)
    nb = jnp.tanh(jax.lax.dot_general(nf, wenc_ref[...],
                                      (((1,), (0,)), ((), ())),
                                      preferred_element_type=jnp.float32)
                  + benc_ref[...])
    pr = pr_ref[...]
    nrep = jnp.sum(pr[:, :, None] * nb.reshape(TAIL_ROWS, K, D), axis=1)
    t = t_ref[...]
    h = jnp.concatenate([t, nrep], axis=1)
    h = jax.lax.dot_general(h, w1_ref[...], (((1,), (0,)), ((), ())),
                            preferred_element_type=jnp.float32) + b1_ref[...]
    pk = jax.nn.sigmoid(
        jax.lax.dot_general(h, w2_ref[...], (((1,), (0,)), ((), ())),
                            preferred_element_type=jnp.float32) + b2_ref[...])
    lab = lab_ref[...]
    cls_iota = jax.lax.broadcasted_iota(jnp.int32, (TAIL_ROWS, NUM_CLASSES), 1)
    acc = jnp.zeros((TAIL_ROWS, NUM_CLASSES), jnp.float32)
    for k in range(K):
        acc = acc + jnp.where(lab[:, k:k + 1] == cls_iota, pr[:, k:k + 1], 0.0)
    out_ref[...] = jnp.log(pk * acc + (1.0 - pk) * mp_ref[...] + 1e-12)


@jax.jit
def _tail(text_rep, probs, labels, nf_g, model_prob, W_enc, b_enc, W1, b1, W2, b2):
    return pl.pallas_call(
        _tail_body,
        grid=(B // TAIL_ROWS,),
        in_specs=[
            pl.BlockSpec((TAIL_ROWS, D), lambda i: (i, 0)),
            pl.BlockSpec((TAIL_ROWS, K), lambda i: (i, 0)),
            pl.BlockSpec((TAIL_ROWS, K), lambda i: (i, 0)),
            pl.BlockSpec((TAIL_ROWS, K, D), lambda i: (i, 0, 0)),
            pl.BlockSpec((TAIL_ROWS, NUM_CLASSES), lambda i: (i, 0)),
            pl.BlockSpec((D, D), lambda i: (0, 0)),
            pl.BlockSpec((1, D), lambda i: (0, 0)),
            pl.BlockSpec((2 * D, 2 * D), lambda i: (0, 0)),
            pl.BlockSpec((1, 2 * D), lambda i: (0, 0)),
            pl.BlockSpec((2 * D, 1), lambda i: (0, 0)),
            pl.BlockSpec((1, 1), lambda i: (0, 0)),
        ],
        out_specs=pl.BlockSpec((TAIL_ROWS, NUM_CLASSES), lambda i: (i, 0)),
        out_shape=jax.ShapeDtypeStruct((B, NUM_CLASSES), jnp.float32),
    )(text_rep, probs, labels, nf_g, model_prob,
      W_enc, b_enc.reshape(1, D), W1, b1.reshape(1, 2 * D), W2,
      b2.reshape(1, 1))


def kernel(x, x_mask, x_idx, keys_store, store_vals, neighbor_feats,
           W_enc, b_enc, W_cls, b_cls, W1, b1, W2, b2):
    text_rep, model_prob = _encode(x, x_mask, W_enc, b_enc, W_cls, b_cls)

    q = keys_store[x_idx]
    scores, gm_t = _scores(q, keys_store, x_idx)
    gm1 = gm_t.transpose(2, 0, 1).reshape(B, G)

    probs, labels, nf_g = _select(gm1, scores.reshape(B * G, 128),
                                  keys_store, neighbor_feats, store_vals,
                                  text_rep)

    return _tail(text_rep, probs, labels, nf_g, model_prob,
                 W_enc, b_enc, W1, b1, W2, b2)


# speedup vs baseline: 29.3222x; 1.0303x over previous
import functools

import jax
import jax.numpy as jnp
from jax import lax
from jax.experimental import pallas as pl
from jax.experimental.pallas import tpu as pltpu, tpu_sc as plsc

B = 128
S = 128
D = 768
K_STORE = 100000
NUM_CLASSES = 1000
K = 32
TEMP = 10.0

TILE_N = 4096
N_PAD = 102400
N_TILES = N_PAD // TILE_N
G = N_PAD // 128
NCHUNK = G // 16

_NEG = -3.0e38


def _score_body(q_ref, k_ref, xidx_ref, s_ref, gm_ref):
    j = pl.program_id(0)
    q = q_ref[...]
    kb = k_ref[...]
    dot = jax.lax.dot_general(q, kb, (((1,), (1,)), ((), ())),
                              preferred_element_type=jnp.float32)
    k2 = jnp.sum(kb * kb, axis=1)
    s = 2.0 * dot - k2[None, :]
    col = jax.lax.broadcasted_iota(jnp.int32, (B, TILE_N), 1) + j * TILE_N
    self_col = xidx_ref[...]
    mask = (col == self_col) | (col >= K_STORE)
    s = jnp.where(mask, _NEG, s)
    s_ref[...] = s
    gm_ref[...] = jnp.max(s.reshape(B, TILE_N // 128, 128), axis=2).transpose(1, 0)[None]


@jax.jit
def _scores(q, keys_pad, x_idx):
    return pl.pallas_call(
        _score_body,
        grid=(N_TILES,),
        in_specs=[
            pl.BlockSpec((B, D), lambda j: (0, 0)),
            pl.BlockSpec((TILE_N, D), lambda j: (j, 0)),
            pl.BlockSpec((B, 1), lambda j: (0, 0)),
        ],
        out_specs=[
            pl.BlockSpec((B, TILE_N), lambda j: (0, j)),
            pl.BlockSpec((1, TILE_N // 128, B), lambda j: (j, 0, 0)),
        ],
        out_shape=[
            jax.ShapeDtypeStruct((B, N_PAD), jnp.float32),
            jax.ShapeDtypeStruct((N_TILES, TILE_N // 128, B), jnp.float32),
        ],
    )(q, keys_pad, x_idx.astype(jnp.int32).reshape(B, 1))


def _merge32(ka, va, kb, vb):
    rb0 = lax.rev(kb[0], (0,))
    rb1 = lax.rev(kb[1], (0,))
    rv0 = lax.rev(vb[0], (0,))
    rv1 = lax.rev(vb[1], (0,))
    ge0 = ka[0] >= rb1
    k0 = jnp.where(ge0, ka[0], rb1)
    v0 = jnp.where(ge0, va[0], rv1)
    ge1 = ka[1] >= rb0
    k1 = jnp.where(ge1, ka[1], rb0)
    v1 = jnp.where(ge1, va[1], rv0)
    geh = k0 >= k1
    hk = jnp.where(geh, k0, k1)
    hv = jnp.where(geh, v0, v1)
    lk = jnp.where(geh, k1, k0)
    lv = jnp.where(geh, v1, v0)
    hk, hv = plsc.sort_key_val(hk, hv, descending=True)
    lk, lv = plsc.sort_key_val(lk, lv, descending=True)
    return (hk, lk), (hv, lv)


def _fold_chunk(acc, k16, v16):
    neg = jnp.full((16,), _NEG, jnp.float32)
    zero = jnp.zeros((16,), jnp.int32)
    sk, sv = plsc.sort_key_val(k16, v16, descending=True)
    if acc is None:
        return (sk, neg), (sv, zero)
    ka, va = acc
    return _merge32(ka, va, (sk, neg), (sv, zero))


def _select_body(gm_hbm, scores_hbm, keys_hbm, nf_hbm, sv_hbm, trep_hbm,
                 probs_hbm, labels_hbm, nfg_hbm,
                 gm_v, cand_v, idx_v, gids_v, knns_v, t_v, krows_v, nfrows_v,
                 labels_v, probs_v, sem):
    nc = 2
    wid = lax.axis_index("s") * nc + lax.axis_index("c")
    iota = lax.iota(jnp.int32, 16)

    def row_body(t, carry):
        r = wid * 4 + t
        pltpu.sync_copy(gm_hbm.at[r], gm_v)

        acc = None
        for j in range(NCHUNK):
            k16 = gm_v[pl.ds(16 * j, 16)]
            v16 = iota + 16 * j
            acc = _fold_chunk(acc, k16, v16)
        gk, gv = acc
        gids_v[pl.ds(0, 16)] = gv[0]
        gids_v[pl.ds(16, 16)] = gv[1]
        idx_v[pl.ds(0, 16)] = gv[0] + r * G
        idx_v[pl.ds(16, 16)] = gv[1] + r * G

        pltpu.async_copy(scores_hbm.at[idx_v], cand_v, sem).wait()

        submax = []
        for g in range(32):
            m = cand_v[g, pl.ds(0, 16)]
            for i in range(1, 8):
                m = jnp.maximum(m, cand_v[g, pl.ds(16 * i, 16)])
            submax.append(m)

        acc = None
        for g in range(32):
            acc = _fold_chunk(acc, submax[g], iota + 16 * g)
        pk, pv = acc

        acc = None
        for half in range(2):
            p = pv[half]
            g = lax.shift_right_logical(p, 4)
            j = jnp.bitwise_and(p, 15)
            gidv = plsc.load_gather(gids_v, [g])
            base = gidv * 128 + j
            for i in range(8):
                k16 = plsc.load_gather(cand_v, [g, j + 16 * i])
                acc = _fold_chunk(acc, k16, base + 16 * i)
        fk, fv = acc
        knns_v[pl.ds(0, 16)] = fv[0]
        knns_v[pl.ds(16, 16)] = fv[1]

        pltpu.sync_copy(trep_hbm.at[r], t_v)
        pltpu.async_copy(keys_hbm.at[knns_v], krows_v, sem).wait()
        pltpu.async_copy(nf_hbm.at[knns_v], nfrows_v, sem).wait()
        pltpu.async_copy(sv_hbm.at[knns_v], labels_v, sem).wait()
        pltpu.sync_copy(nfrows_v, nfg_hbm.at[r])
        pltpu.sync_copy(labels_v, labels_hbm.at[r])

        def dist_body(n, dcarry):
            d0, d1 = dcarry
            acc = jnp.zeros((16,), jnp.float32)
            for i in range(D // 16):
                diff = t_v[pl.ds(16 * i, 16)] - krows_v[n, pl.ds(16 * i, 16)]
                acc = acc + diff * diff
            dn = jnp.sum(acc)
            lane = jnp.bitwise_and(n, 15)
            dsplat = jnp.full((16,), dn, jnp.float32)
            d0 = jnp.where((iota == lane) & (n < 16), dsplat, d0)
            d1 = jnp.where((iota == lane) & (n >= 16), dsplat, d1)
            return d0, d1

        zero16 = jnp.zeros((16,), jnp.float32)
        d0, d1 = lax.fori_loop(0, K, dist_body, (zero16, zero16))

        u0 = jnp.negative(d0) / TEMP
        u1 = jnp.negative(d1) / TEMP
        m = jnp.maximum(jnp.max(u0), jnp.max(u1))
        msp = jnp.full((16,), m, jnp.float32)
        e0 = jnp.exp(u0 - msp)
        e1 = jnp.exp(u1 - msp)
        ssum = jnp.sum(e0) + jnp.sum(e1)
        ssp = jnp.full((16,), ssum, jnp.float32)
        probs_v[pl.ds(0, 16)] = e0 / ssp
        probs_v[pl.ds(16, 16)] = e1 / ssp
        pltpu.sync_copy(probs_v, probs_hbm.at[r])
        return carry

    lax.fori_loop(0, 4, row_body, 0)


@jax.jit
def _select(gm1, scores_resh, keys_store, neighbor_feats, store_vals, text_rep):
    mesh = plsc.VectorSubcoreMesh(core_axis_name="c", subcore_axis_name="s")
    return pl.kernel(
        _select_body,
        mesh=mesh,
        out_type=[
            jax.ShapeDtypeStruct((B, K), jnp.float32),
            jax.ShapeDtypeStruct((B, K), jnp.int32),
            jax.ShapeDtypeStruct((B, K, D), jnp.float32),
        ],
        scratch_types=[
            pltpu.VMEM((G,), jnp.float32),
            pltpu.VMEM((32, 128), jnp.float32),
            pltpu.VMEM((32,), jnp.int32),
            pltpu.VMEM((32,), jnp.int32),
            pltpu.VMEM((32,), jnp.int32),
            pltpu.VMEM((D,), jnp.float32),
            pltpu.VMEM((K, D), jnp.float32),
            pltpu.VMEM((K, D), jnp.float32),
            pltpu.VMEM((K,), jnp.int32),
            pltpu.VMEM((K,), jnp.float32),
            pltpu.SemaphoreType.DMA,
        ],
        compiler_params=pltpu.CompilerParams(needs_layout_passes=False),
    )(gm1, scores_resh, keys_store, neighbor_feats, store_vals, text_rep)


ENC_ROWS = 32


def _encode_body(x_ref, m_ref, wenc_ref, benc_ref, wcls_ref, bcls_ref,
                 t_ref, p_ref):
    xm = x_ref[...] * m_ref[...][:, :, None]
    pooled = jnp.sum(xm, axis=1) / jnp.maximum(
        jnp.sum(m_ref[...], axis=1), 1.0)[:, None]
    t = jnp.tanh(jax.lax.dot_general(pooled, wenc_ref[...],
                                     (((1,), (0,)), ((), ())),
                                     preferred_element_type=jnp.float32)
                 + benc_ref[...])
    logits = jax.lax.dot_general(t, wcls_ref[...], (((1,), (0,)), ((), ())),
                                 preferred_element_type=jnp.float32) + bcls_ref[...]
    mx = jnp.max(logits, axis=1, keepdims=True)
    e = jnp.exp(logits - mx)
    t_ref[...] = t
    p_ref[...] = e / jnp.sum(e, axis=1, keepdims=True)


@jax.jit
def _encode(x, x_mask, W_enc, b_enc, W_cls, b_cls):
    return pl.pallas_call(
        _encode_body,
        grid=(B // ENC_ROWS,),
        in_specs=[
            pl.BlockSpec((ENC_ROWS, S, D), lambda i: (i, 0, 0)),
            pl.BlockSpec((ENC_ROWS, S), lambda i: (i, 0)),
            pl.BlockSpec((D, D), lambda i: (0, 0)),
            pl.BlockSpec((1, D), lambda i: (0, 0)),
            pl.BlockSpec((D, NUM_CLASSES), lambda i: (0, 0)),
            pl.BlockSpec((1, NUM_CLASSES), lambda i: (0, 0)),
        ],
        out_specs=[
            pl.BlockSpec((ENC_ROWS, D), lambda i: (i, 0)),
            pl.BlockSpec((ENC_ROWS, NUM_CLASSES), lambda i: (i, 0)),
        ],
        out_shape=[
            jax.ShapeDtypeStruct((B, D), jnp.float32),
            jax.ShapeDtypeStruct((B, NUM_CLASSES), jnp.float32),
        ],
    )(x, x_mask, W_enc, b_enc.reshape(1, D), W_cls, b_cls.reshape(1, NUM_CLASSES))


TAIL_ROWS = 32


def _tail_body(t_ref, pr_ref, lab_ref, nfg_ref, mp_ref,
               wenc_ref, benc_ref, w1_ref, b1_ref, w2_ref, b2_ref, out_ref):
    nf = nfg_ref[...].reshape(TAIL_ROWS * K, D)
    nb = jnp.tanh(jax.lax.dot_general(nf, wenc_ref[...],
                                      (((1,), (0,)), ((), ())),
                                      preferred_element_type=jnp.float32)
                  + benc_ref[...])
    pr = pr_ref[...]
    nrep = jnp.sum(pr[:, :, None] * nb.reshape(TAIL_ROWS, K, D), axis=1)
    t = t_ref[...]
    h = jnp.concatenate([t, nrep], axis=1)
    h = jax.lax.dot_general(h, w1_ref[...], (((1,), (0,)), ((), ())),
                            preferred_element_type=jnp.float32) + b1_ref[...]
    pk = jax.nn.sigmoid(
        jax.lax.dot_general(h, w2_ref[...], (((1,), (0,)), ((), ())),
                            preferred_element_type=jnp.float32) + b2_ref[...])
    lab = lab_ref[...]
    cls_iota = jax.lax.broadcasted_iota(jnp.int32, (TAIL_ROWS, NUM_CLASSES), 1)
    acc = jnp.zeros((TAIL_ROWS, NUM_CLASSES), jnp.float32)
    for k in range(K):
        acc = acc + jnp.where(lab[:, k:k + 1] == cls_iota, pr[:, k:k + 1], 0.0)
    out_ref[...] = jnp.log(pk * acc + (1.0 - pk) * mp_ref[...] + 1e-12)


@jax.jit
def _tail(text_rep, probs, labels, nf_g, model_prob, W_enc, b_enc, W1, b1, W2, b2):
    return pl.pallas_call(
        _tail_body,
        grid=(B // TAIL_ROWS,),
        in_specs=[
            pl.BlockSpec((TAIL_ROWS, D), lambda i: (i, 0)),
            pl.BlockSpec((TAIL_ROWS, K), lambda i: (i, 0)),
            pl.BlockSpec((TAIL_ROWS, K), lambda i: (i, 0)),
            pl.BlockSpec((TAIL_ROWS, K, D), lambda i: (i, 0, 0)),
            pl.BlockSpec((TAIL_ROWS, NUM_CLASSES), lambda i: (i, 0)),
            pl.BlockSpec((D, D), lambda i: (0, 0)),
            pl.BlockSpec((1, D), lambda i: (0, 0)),
            pl.BlockSpec((2 * D, 2 * D), lambda i: (0, 0)),
            pl.BlockSpec((1, 2 * D), lambda i: (0, 0)),
            pl.BlockSpec((2 * D, 1), lambda i: (0, 0)),
            pl.BlockSpec((1, 1), lambda i: (0, 0)),
        ],
        out_specs=pl.BlockSpec((TAIL_ROWS, NUM_CLASSES), lambda i: (i, 0)),
        out_shape=jax.ShapeDtypeStruct((B, NUM_CLASSES), jnp.float32),
    )(text_rep, probs, labels, nf_g, model_prob,
      W_enc, b_enc.reshape(1, D), W1, b1.reshape(1, 2 * D), W2,
      b2.reshape(1, 1))


def kernel(x, x_mask, x_idx, keys_store, store_vals, neighbor_feats,
           W_enc, b_enc, W_cls, b_cls, W1, b1, W2, b2):
    text_rep, model_prob = _encode(x, x_mask, W_enc, b_enc, W_cls, b_cls)

    q = keys_store[x_idx]
    scores, gm_t = _scores(q, keys_store, x_idx)
    gm1 = gm_t.transpose(2, 0, 1).reshape(B, G)

    probs, labels, nf_g = _select(gm1, scores.reshape(B * G, 128),
                                  keys_store, neighbor_feats, store_vals,
                                  text_rep)

    return _tail(text_rep, probs, labels, nf_g, model_prob,
                 W_enc, b_enc, W1, b1, W2, b2)


# speedup vs baseline: 31.9800x; 1.0906x over previous
import functools

import jax
import jax.numpy as jnp
from jax import lax
from jax.experimental import pallas as pl
from jax.experimental.pallas import tpu as pltpu, tpu_sc as plsc

B = 128
S = 128
D = 768
K_STORE = 100000
NUM_CLASSES = 1000
K = 32
TEMP = 10.0

TILE_N = 4096
N_PAD = 102400
N_TILES = N_PAD // TILE_N
G = N_PAD // 128
NCHUNK = G // 16

_NEG = -3.0e38


def _score_body(q_ref, k_ref, xidx_ref, s_ref, gm_ref):
    j = pl.program_id(0)
    q = q_ref[...]
    kb = k_ref[...]
    dot = jax.lax.dot_general(q, kb, (((1,), (1,)), ((), ())),
                              preferred_element_type=jnp.float32)
    k2 = jnp.sum(kb * kb, axis=1)
    s = 2.0 * dot - k2[None, :]
    col = jax.lax.broadcasted_iota(jnp.int32, (B, TILE_N), 1) + j * TILE_N
    self_col = xidx_ref[...]
    mask = (col == self_col) | (col >= K_STORE)
    s = jnp.where(mask, _NEG, s)
    s_ref[...] = s
    gm_ref[...] = jnp.max(s.reshape(B, TILE_N // 128, 128), axis=2).transpose(1, 0)[None]


@jax.jit
def _scores(q, keys_pad, x_idx):
    return pl.pallas_call(
        _score_body,
        grid=(N_TILES,),
        in_specs=[
            pl.BlockSpec((B, D), lambda j: (0, 0)),
            pl.BlockSpec((TILE_N, D), lambda j: (j, 0)),
            pl.BlockSpec((B, 1), lambda j: (0, 0)),
        ],
        out_specs=[
            pl.BlockSpec((B, TILE_N), lambda j: (0, j)),
            pl.BlockSpec((1, TILE_N // 128, B), lambda j: (j, 0, 0)),
        ],
        out_shape=[
            jax.ShapeDtypeStruct((B, N_PAD), jnp.float32),
            jax.ShapeDtypeStruct((N_TILES, TILE_N // 128, B), jnp.float32),
        ],
    )(q, keys_pad, x_idx.astype(jnp.int32).reshape(B, 1))


def _merge32(ka, va, kb, vb):
    rb0 = lax.rev(kb[0], (0,))
    rb1 = lax.rev(kb[1], (0,))
    rv0 = lax.rev(vb[0], (0,))
    rv1 = lax.rev(vb[1], (0,))
    ge0 = ka[0] >= rb1
    k0 = jnp.where(ge0, ka[0], rb1)
    v0 = jnp.where(ge0, va[0], rv1)
    ge1 = ka[1] >= rb0
    k1 = jnp.where(ge1, ka[1], rb0)
    v1 = jnp.where(ge1, va[1], rv0)
    geh = k0 >= k1
    hk = jnp.where(geh, k0, k1)
    hv = jnp.where(geh, v0, v1)
    lk = jnp.where(geh, k1, k0)
    lv = jnp.where(geh, v1, v0)
    hk, hv = plsc.sort_key_val(hk, hv, descending=True)
    lk, lv = plsc.sort_key_val(lk, lv, descending=True)
    return (hk, lk), (hv, lv)


def _fold_chunk(acc, k16, v16):
    neg = jnp.full((16,), _NEG, jnp.float32)
    zero = jnp.zeros((16,), jnp.int32)
    sk, sv = plsc.sort_key_val(k16, v16, descending=True)
    if acc is None:
        return (sk, neg), (sv, zero)
    ka, va = acc
    return _merge32(ka, va, (sk, neg), (sv, zero))


def _select_body(gm_hbm, scores_hbm, keys_hbm, nf_hbm, sv_hbm,
                 labels_hbm, knnk_hbm, nfg_hbm,
                 gm_v, cand_v, idx_v, gids_v, knns_v, krows_v, nfrows_v,
                 labels_v, sem):
    nc = 2
    wid = lax.axis_index("s") * nc + lax.axis_index("c")
    iota = lax.iota(jnp.int32, 16)

    def row_body(t, carry):
        r = wid * 4 + t
        pltpu.sync_copy(gm_hbm.at[r], gm_v)

        acc = None
        for j in range(NCHUNK):
            k16 = gm_v[pl.ds(16 * j, 16)]
            v16 = iota + 16 * j
            acc = _fold_chunk(acc, k16, v16)
        gk, gv = acc
        gids_v[pl.ds(0, 16)] = gv[0]
        gids_v[pl.ds(16, 16)] = gv[1]
        idx_v[pl.ds(0, 16)] = gv[0] + r * G
        idx_v[pl.ds(16, 16)] = gv[1] + r * G

        pltpu.async_copy(scores_hbm.at[idx_v], cand_v, sem).wait()

        submax = []
        for g in range(32):
            m = cand_v[g, pl.ds(0, 16)]
            for i in range(1, 8):
                m = jnp.maximum(m, cand_v[g, pl.ds(16 * i, 16)])
            submax.append(m)

        acc = None
        for g in range(32):
            acc = _fold_chunk(acc, submax[g], iota + 16 * g)
        pk, pv = acc

        acc = None
        for half in range(2):
            p = pv[half]
            g = lax.shift_right_logical(p, 4)
            j = jnp.bitwise_and(p, 15)
            gidv = plsc.load_gather(gids_v, [g])
            base = gidv * 128 + j
            for i in range(8):
                k16 = plsc.load_gather(cand_v, [g, j + 16 * i])
                acc = _fold_chunk(acc, k16, base + 16 * i)
        fk, fv = acc
        knns_v[pl.ds(0, 16)] = fv[0]
        knns_v[pl.ds(16, 16)] = fv[1]

        ck = pltpu.async_copy(keys_hbm.at[knns_v], krows_v, sem)
        cn = pltpu.async_copy(nf_hbm.at[knns_v], nfrows_v, sem)
        cl = pltpu.async_copy(sv_hbm.at[knns_v], labels_v, sem)
        ck.wait()
        cn.wait()
        cl.wait()
        pltpu.sync_copy(krows_v, knnk_hbm.at[r])
        pltpu.sync_copy(nfrows_v, nfg_hbm.at[r])
        pltpu.sync_copy(labels_v, labels_hbm.at[r])
        return carry

    lax.fori_loop(0, 4, row_body, 0)


@jax.jit
def _select(gm1, scores_resh, keys_store, neighbor_feats, store_vals):
    mesh = plsc.VectorSubcoreMesh(core_axis_name="c", subcore_axis_name="s")
    return pl.kernel(
        _select_body,
        mesh=mesh,
        out_type=[
            jax.ShapeDtypeStruct((B, K), jnp.int32),
            jax.ShapeDtypeStruct((B, K, D), jnp.float32),
            jax.ShapeDtypeStruct((B, K, D), jnp.float32),
        ],
        scratch_types=[
            pltpu.VMEM((G,), jnp.float32),
            pltpu.VMEM((32, 128), jnp.float32),
            pltpu.VMEM((32,), jnp.int32),
            pltpu.VMEM((32,), jnp.int32),
            pltpu.VMEM((32,), jnp.int32),
            pltpu.VMEM((K, D), jnp.float32),
            pltpu.VMEM((K, D), jnp.float32),
            pltpu.VMEM((K,), jnp.int32),
            pltpu.SemaphoreType.DMA,
        ],
        compiler_params=pltpu.CompilerParams(needs_layout_passes=False),
    )(gm1, scores_resh, keys_store, neighbor_feats, store_vals)


ENC_ROWS = 32


def _encode_body(x_ref, m_ref, wenc_ref, benc_ref, wcls_ref, bcls_ref,
                 t_ref, p_ref):
    xm = x_ref[...] * m_ref[...][:, :, None]
    pooled = jnp.sum(xm, axis=1) / jnp.maximum(
        jnp.sum(m_ref[...], axis=1), 1.0)[:, None]
    t = jnp.tanh(jax.lax.dot_general(pooled, wenc_ref[...],
                                     (((1,), (0,)), ((), ())),
                                     preferred_element_type=jnp.float32)
                 + benc_ref[...])
    logits = jax.lax.dot_general(t, wcls_ref[...], (((1,), (0,)), ((), ())),
                                 preferred_element_type=jnp.float32) + bcls_ref[...]
    mx = jnp.max(logits, axis=1, keepdims=True)
    e = jnp.exp(logits - mx)
    t_ref[...] = t
    p_ref[...] = e / jnp.sum(e, axis=1, keepdims=True)


@jax.jit
def _encode(x, x_mask, W_enc, b_enc, W_cls, b_cls):
    return pl.pallas_call(
        _encode_body,
        grid=(B // ENC_ROWS,),
        in_specs=[
            pl.BlockSpec((ENC_ROWS, S, D), lambda i: (i, 0, 0)),
            pl.BlockSpec((ENC_ROWS, S), lambda i: (i, 0)),
            pl.BlockSpec((D, D), lambda i: (0, 0)),
            pl.BlockSpec((1, D), lambda i: (0, 0)),
            pl.BlockSpec((D, NUM_CLASSES), lambda i: (0, 0)),
            pl.BlockSpec((1, NUM_CLASSES), lambda i: (0, 0)),
        ],
        out_specs=[
            pl.BlockSpec((ENC_ROWS, D), lambda i: (i, 0)),
            pl.BlockSpec((ENC_ROWS, NUM_CLASSES), lambda i: (i, 0)),
        ],
        out_shape=[
            jax.ShapeDtypeStruct((B, D), jnp.float32),
            jax.ShapeDtypeStruct((B, NUM_CLASSES), jnp.float32),
        ],
    )(x, x_mask, W_enc, b_enc.reshape(1, D), W_cls, b_cls.reshape(1, NUM_CLASSES))


TAIL_ROWS = 32


def _tail_body(t_ref, lab_ref, knnk_ref, nfg_ref, mp_ref,
               wenc_ref, benc_ref, w1_ref, b1_ref, w2_ref, b2_ref, out_ref):
    t = t_ref[...]
    dists = jnp.sum((t[:, None, :] - knnk_ref[...]) ** 2, axis=-1)
    u = -dists / TEMP
    um = jnp.max(u, axis=1, keepdims=True)
    e = jnp.exp(u - um)
    pr = e / jnp.sum(e, axis=1, keepdims=True)

    nf = nfg_ref[...].reshape(TAIL_ROWS * K, D)
    nb = jnp.tanh(jax.lax.dot_general(nf, wenc_ref[...],
                                      (((1,), (0,)), ((), ())),
                                      preferred_element_type=jnp.float32)
                  + benc_ref[...])
    nrep = jnp.sum(pr[:, :, None] * nb.reshape(TAIL_ROWS, K, D), axis=1)
    h = jnp.concatenate([t, nrep], axis=1)
    h = jax.lax.dot_general(h, w1_ref[...], (((1,), (0,)), ((), ())),
                            preferred_element_type=jnp.float32) + b1_ref[...]
    pk = jax.nn.sigmoid(
        jax.lax.dot_general(h, w2_ref[...], (((1,), (0,)), ((), ())),
                            preferred_element_type=jnp.float32) + b2_ref[...])
    lab = lab_ref[...]
    cls_iota = jax.lax.broadcasted_iota(jnp.int32, (TAIL_ROWS, NUM_CLASSES), 1)
    acc = jnp.zeros((TAIL_ROWS, NUM_CLASSES), jnp.float32)
    for k in range(K):
        acc = acc + jnp.where(lab[:, k:k + 1] == cls_iota, pr[:, k:k + 1], 0.0)
    out_ref[...] = jnp.log(pk * acc + (1.0 - pk) * mp_ref[...] + 1e-12)


@jax.jit
def _tail(text_rep, labels, knnk, nf_g, model_prob, W_enc, b_enc, W1, b1, W2, b2):
    return pl.pallas_call(
        _tail_body,
        grid=(B // TAIL_ROWS,),
        in_specs=[
            pl.BlockSpec((TAIL_ROWS, D), lambda i: (i, 0)),
            pl.BlockSpec((TAIL_ROWS, K), lambda i: (i, 0)),
            pl.BlockSpec((TAIL_ROWS, K, D), lambda i: (i, 0, 0)),
            pl.BlockSpec((TAIL_ROWS, K, D), lambda i: (i, 0, 0)),
            pl.BlockSpec((TAIL_ROWS, NUM_CLASSES), lambda i: (i, 0)),
            pl.BlockSpec((D, D), lambda i: (0, 0)),
            pl.BlockSpec((1, D), lambda i: (0, 0)),
            pl.BlockSpec((2 * D, 2 * D), lambda i: (0, 0)),
            pl.BlockSpec((1, 2 * D), lambda i: (0, 0)),
            pl.BlockSpec((2 * D, 1), lambda i: (0, 0)),
            pl.BlockSpec((1, 1), lambda i: (0, 0)),
        ],
        out_specs=pl.BlockSpec((TAIL_ROWS, NUM_CLASSES), lambda i: (i, 0)),
        out_shape=jax.ShapeDtypeStruct((B, NUM_CLASSES), jnp.float32),
    )(text_rep, labels, knnk, nf_g, model_prob,
      W_enc, b_enc.reshape(1, D), W1, b1.reshape(1, 2 * D), W2,
      b2.reshape(1, 1))


def kernel(x, x_mask, x_idx, keys_store, store_vals, neighbor_feats,
           W_enc, b_enc, W_cls, b_cls, W1, b1, W2, b2):
    q = keys_store[x_idx]
    scores, gm_t = _scores(q, keys_store, x_idx)
    gm1 = gm_t.transpose(2, 0, 1).reshape(B, G)

    labels, knnk, nf_g = _select(gm1, scores.reshape(B * G, 128),
                                 keys_store, neighbor_feats, store_vals)

    text_rep, model_prob = _encode(x, x_mask, W_enc, b_enc, W_cls, b_cls)

    return _tail(text_rep, labels, knnk, nf_g, model_prob,
                 W_enc, b_enc, W1, b1, W2, b2)


# speedup vs baseline: 31.9875x; 1.0002x over previous
import functools

import jax
import jax.numpy as jnp
from jax import lax
from jax.experimental import pallas as pl
from jax.experimental.pallas import tpu as pltpu, tpu_sc as plsc

B = 128
S = 128
D = 768
K_STORE = 100000
NUM_CLASSES = 1000
K = 32
TEMP = 10.0

TILE_N = 4096
N_PAD = 102400
N_TILES = N_PAD // TILE_N
G = N_PAD // 128
NCHUNK = G // 16

_NEG = -3.0e38


def _score_body(q_ref, k_ref, xidx_ref, s_ref, gm_ref):
    j = pl.program_id(0)
    q = q_ref[...]
    kb = k_ref[...]
    dot = jax.lax.dot_general(q, kb, (((1,), (1,)), ((), ())),
                              preferred_element_type=jnp.float32)
    k2 = jnp.sum(kb * kb, axis=1)
    s = 2.0 * dot - k2[None, :]
    col = jax.lax.broadcasted_iota(jnp.int32, (B, TILE_N), 1) + j * TILE_N
    self_col = xidx_ref[...]
    mask = (col == self_col) | (col >= K_STORE)
    s = jnp.where(mask, _NEG, s)
    s_ref[...] = s
    gm_ref[...] = jnp.max(s.reshape(B, TILE_N // 128, 128), axis=2).transpose(1, 0)[None]


@jax.jit
def _scores(q, keys_pad, x_idx):
    return pl.pallas_call(
        _score_body,
        grid=(N_TILES,),
        in_specs=[
            pl.BlockSpec((B, D), lambda j: (0, 0)),
            pl.BlockSpec((TILE_N, D), lambda j: (j, 0)),
            pl.BlockSpec((B, 1), lambda j: (0, 0)),
        ],
        out_specs=[
            pl.BlockSpec((B, TILE_N), lambda j: (0, j)),
            pl.BlockSpec((1, TILE_N // 128, B), lambda j: (j, 0, 0)),
        ],
        out_shape=[
            jax.ShapeDtypeStruct((B, N_PAD), jnp.float32),
            jax.ShapeDtypeStruct((N_TILES, TILE_N // 128, B), jnp.float32),
        ],
    )(q, keys_pad, x_idx.astype(jnp.int32).reshape(B, 1))


def _merge32(ka, va, kb, vb):
    rb0 = lax.rev(kb[0], (0,))
    rb1 = lax.rev(kb[1], (0,))
    rv0 = lax.rev(vb[0], (0,))
    rv1 = lax.rev(vb[1], (0,))
    ge0 = ka[0] >= rb1
    k0 = jnp.where(ge0, ka[0], rb1)
    v0 = jnp.where(ge0, va[0], rv1)
    ge1 = ka[1] >= rb0
    k1 = jnp.where(ge1, ka[1], rb0)
    v1 = jnp.where(ge1, va[1], rv0)
    geh = k0 >= k1
    hk = jnp.where(geh, k0, k1)
    hv = jnp.where(geh, v0, v1)
    lk = jnp.where(geh, k1, k0)
    lv = jnp.where(geh, v1, v0)
    hk, hv = plsc.sort_key_val(hk, hv, descending=True)
    lk, lv = plsc.sort_key_val(lk, lv, descending=True)
    return (hk, lk), (hv, lv)


def _fold_chunk(acc, k16, v16):
    neg = jnp.full((16,), _NEG, jnp.float32)
    zero = jnp.zeros((16,), jnp.int32)
    sk, sv = plsc.sort_key_val(k16, v16, descending=True)
    if acc is None:
        return (sk, neg), (sv, zero)
    ka, va = acc
    return _merge32(ka, va, (sk, neg), (sv, zero))


def _select_body(gm_hbm, scores_hbm, keys_hbm, nf_hbm, sv_hbm,
                 labels_hbm, knnk_hbm, nfg_hbm,
                 gm_v, cand_v, idx_v, gids_v, knns_v, krows_v, nfrows_v,
                 labels_v, sem):
    nc = 2
    wid = lax.axis_index("s") * nc + lax.axis_index("c")
    iota = lax.iota(jnp.int32, 16)

    def row_body(t, carry):
        r = wid * 4 + t
        pltpu.sync_copy(gm_hbm.at[r], gm_v)

        acc = None
        for j in range(NCHUNK):
            k16 = gm_v[pl.ds(16 * j, 16)]
            v16 = iota + 16 * j
            acc = _fold_chunk(acc, k16, v16)
        gk, gv = acc
        gids_v[pl.ds(0, 16)] = gv[0]
        gids_v[pl.ds(16, 16)] = gv[1]
        idx_v[pl.ds(0, 16)] = gv[0] + r * G
        idx_v[pl.ds(16, 16)] = gv[1] + r * G

        pltpu.async_copy(scores_hbm.at[idx_v], cand_v, sem).wait()

        submax = []
        for g in range(32):
            m = cand_v[g, pl.ds(0, 16)]
            for i in range(1, 8):
                m = jnp.maximum(m, cand_v[g, pl.ds(16 * i, 16)])
            submax.append(m)

        acc = None
        for g in range(32):
            acc = _fold_chunk(acc, submax[g], iota + 16 * g)
        pk, pv = acc

        acc = None
        for half in range(2):
            p = pv[half]
            g = lax.shift_right_logical(p, 4)
            j = jnp.bitwise_and(p, 15)
            gidv = plsc.load_gather(gids_v, [g])
            base = gidv * 128 + j
            for i in range(8):
                k16 = plsc.load_gather(cand_v, [g, j + 16 * i])
                acc = _fold_chunk(acc, k16, base + 16 * i)
        fk, fv = acc
        knns_v[pl.ds(0, 16)] = fv[0]
        knns_v[pl.ds(16, 16)] = fv[1]

        ck = pltpu.async_copy(keys_hbm.at[knns_v], krows_v, sem)
        cn = pltpu.async_copy(nf_hbm.at[knns_v], nfrows_v, sem)
        cl = pltpu.async_copy(sv_hbm.at[knns_v], labels_v, sem)
        ck.wait()
        cn.wait()
        cl.wait()
        pltpu.sync_copy(krows_v, knnk_hbm.at[r])
        pltpu.sync_copy(nfrows_v, nfg_hbm.at[r])
        pltpu.sync_copy(labels_v, labels_hbm.at[r])
        return carry

    lax.fori_loop(0, 4, row_body, 0)


@jax.jit
def _select(gm1, scores_resh, keys_store, neighbor_feats, store_vals):
    mesh = plsc.VectorSubcoreMesh(core_axis_name="c", subcore_axis_name="s")
    return pl.kernel(
        _select_body,
        mesh=mesh,
        out_type=[
            jax.ShapeDtypeStruct((B, K), jnp.int32),
            jax.ShapeDtypeStruct((B, K, D), jnp.float32),
            jax.ShapeDtypeStruct((B, K, D), jnp.float32),
        ],
        scratch_types=[
            pltpu.VMEM((G,), jnp.float32),
            pltpu.VMEM((32, 128), jnp.float32),
            pltpu.VMEM((32,), jnp.int32),
            pltpu.VMEM((32,), jnp.int32),
            pltpu.VMEM((32,), jnp.int32),
            pltpu.VMEM((K, D), jnp.float32),
            pltpu.VMEM((K, D), jnp.float32),
            pltpu.VMEM((K,), jnp.int32),
            pltpu.SemaphoreType.DMA,
        ],
        compiler_params=pltpu.CompilerParams(needs_layout_passes=False),
    )(gm1, scores_resh, keys_store, neighbor_feats, store_vals)


ENC_ROWS = 32


def _encode_body(x_ref, m_ref, wenc_ref, benc_ref, wcls_ref, bcls_ref,
                 t_ref, p_ref):
    xm = x_ref[...] * m_ref[...][:, :, None]
    pooled = jnp.sum(xm, axis=1) / jnp.maximum(
        jnp.sum(m_ref[...], axis=1), 1.0)[:, None]
    t = jnp.tanh(jax.lax.dot_general(pooled, wenc_ref[...],
                                     (((1,), (0,)), ((), ())),
                                     preferred_element_type=jnp.float32)
                 + benc_ref[...])
    logits = jax.lax.dot_general(t, wcls_ref[...], (((1,), (0,)), ((), ())),
                                 preferred_element_type=jnp.float32) + bcls_ref[...]
    mx = jnp.max(logits, axis=1, keepdims=True)
    e = jnp.exp(logits - mx)
    t_ref[...] = t
    p_ref[...] = e / jnp.sum(e, axis=1, keepdims=True)


@jax.jit
def _encode(x, x_mask, W_enc, b_enc, W_cls, b_cls):
    return pl.pallas_call(
        _encode_body,
        grid=(B // ENC_ROWS,),
        in_specs=[
            pl.BlockSpec((ENC_ROWS, S, D), lambda i: (i, 0, 0)),
            pl.BlockSpec((ENC_ROWS, S), lambda i: (i, 0)),
            pl.BlockSpec((D, D), lambda i: (0, 0)),
            pl.BlockSpec((1, D), lambda i: (0, 0)),
            pl.BlockSpec((D, NUM_CLASSES), lambda i: (0, 0)),
            pl.BlockSpec((1, NUM_CLASSES), lambda i: (0, 0)),
        ],
        out_specs=[
            pl.BlockSpec((ENC_ROWS, D), lambda i: (i, 0)),
            pl.BlockSpec((ENC_ROWS, NUM_CLASSES), lambda i: (i, 0)),
        ],
        out_shape=[
            jax.ShapeDtypeStruct((B, D), jnp.float32),
            jax.ShapeDtypeStruct((B, NUM_CLASSES), jnp.float32),
        ],
    )(x, x_mask, W_enc, b_enc.reshape(1, D), W_cls, b_cls.reshape(1, NUM_CLASSES))


TAIL_ROWS = 32


def _tail_body(t_ref, lab_ref, knnk_ref, nfg_ref, mp_ref,
               wenc_ref, benc_ref, w1_ref, b1_ref, w2_ref, b2_ref, out_ref):
    t = t_ref[...]
    dists = jnp.sum((t[:, None, :] - knnk_ref[...]) ** 2, axis=-1)
    u = -dists / TEMP
    um = jnp.max(u, axis=1, keepdims=True)
    e = jnp.exp(u - um)
    pr = e / jnp.sum(e, axis=1, keepdims=True)

    nf = nfg_ref[...].reshape(TAIL_ROWS * K, D).astype(jnp.bfloat16)
    nb = jnp.tanh(jax.lax.dot_general(nf, wenc_ref[...].astype(jnp.bfloat16),
                                      (((1,), (0,)), ((), ())),
                                      preferred_element_type=jnp.float32)
                  + benc_ref[...])
    nrep = jnp.sum(pr[:, :, None] * nb.reshape(TAIL_ROWS, K, D), axis=1)
    h = jnp.concatenate([t, nrep], axis=1)
    h = jax.lax.dot_general(h, w1_ref[...], (((1,), (0,)), ((), ())),
                            preferred_element_type=jnp.float32) + b1_ref[...]
    pk = jax.nn.sigmoid(
        jax.lax.dot_general(h, w2_ref[...], (((1,), (0,)), ((), ())),
                            preferred_element_type=jnp.float32) + b2_ref[...])
    lab = lab_ref[...]
    cls_iota = jax.lax.broadcasted_iota(jnp.int32, (TAIL_ROWS, NUM_CLASSES), 1)
    acc = jnp.zeros((TAIL_ROWS, NUM_CLASSES), jnp.float32)
    for k in range(K):
        acc = acc + jnp.where(lab[:, k:k + 1] == cls_iota, pr[:, k:k + 1], 0.0)
    out_ref[...] = jnp.log(pk * acc + (1.0 - pk) * mp_ref[...] + 1e-12)


@jax.jit
def _tail(text_rep, labels, knnk, nf_g, model_prob, W_enc, b_enc, W1, b1, W2, b2):
    return pl.pallas_call(
        _tail_body,
        grid=(B // TAIL_ROWS,),
        in_specs=[
            pl.BlockSpec((TAIL_ROWS, D), lambda i: (i, 0)),
            pl.BlockSpec((TAIL_ROWS, K), lambda i: (i, 0)),
            pl.BlockSpec((TAIL_ROWS, K, D), lambda i: (i, 0, 0)),
            pl.BlockSpec((TAIL_ROWS, K, D), lambda i: (i, 0, 0)),
            pl.BlockSpec((TAIL_ROWS, NUM_CLASSES), lambda i: (i, 0)),
            pl.BlockSpec((D, D), lambda i: (0, 0)),
            pl.BlockSpec((1, D), lambda i: (0, 0)),
            pl.BlockSpec((2 * D, 2 * D), lambda i: (0, 0)),
            pl.BlockSpec((1, 2 * D), lambda i: (0, 0)),
            pl.BlockSpec((2 * D, 1), lambda i: (0, 0)),
            pl.BlockSpec((1, 1), lambda i: (0, 0)),
        ],
        out_specs=pl.BlockSpec((TAIL_ROWS, NUM_CLASSES), lambda i: (i, 0)),
        out_shape=jax.ShapeDtypeStruct((B, NUM_CLASSES), jnp.float32),
    )(text_rep, labels, knnk, nf_g, model_prob,
      W_enc, b_enc.reshape(1, D), W1, b1.reshape(1, 2 * D), W2,
      b2.reshape(1, 1))


def kernel(x, x_mask, x_idx, keys_store, store_vals, neighbor_feats,
           W_enc, b_enc, W_cls, b_cls, W1, b1, W2, b2):
    q = keys_store[x_idx]
    scores, gm_t = _scores(q, keys_store, x_idx)
    gm1 = gm_t.transpose(2, 0, 1).reshape(B, G)

    labels, knnk, nf_g = _select(gm1, scores.reshape(B * G, 128),
                                 keys_store, neighbor_feats, store_vals)

    text_rep, model_prob = _encode(x, x_mask, W_enc, b_enc, W_cls, b_cls)

    return _tail(text_rep, labels, knnk, nf_g, model_prob,
                 W_enc, b_enc, W1, b1, W2, b2)


# speedup vs baseline: 31.9915x; 1.0001x over previous
import functools

import jax
import jax.numpy as jnp
from jax import lax
from jax.experimental import pallas as pl
from jax.experimental.pallas import tpu as pltpu, tpu_sc as plsc

B = 128
S = 128
D = 768
K_STORE = 100000
NUM_CLASSES = 1000
K = 32
TEMP = 10.0

TILE_N = 4096
N_PAD = 102400
N_TILES = N_PAD // TILE_N
G = N_PAD // 128
NCHUNK = G // 16

_NEG = -3.0e38


def _score_body(q_ref, k_ref, xidx_ref, s_ref, gm_ref):
    j = pl.program_id(0)
    q = q_ref[...]
    kb = k_ref[...]
    dot = jax.lax.dot_general(q, kb, (((1,), (1,)), ((), ())),
                              preferred_element_type=jnp.float32)
    k2 = jnp.sum(kb * kb, axis=1)
    s = 2.0 * dot - k2[None, :]
    col = jax.lax.broadcasted_iota(jnp.int32, (B, TILE_N), 1) + j * TILE_N
    self_col = xidx_ref[...]
    mask = (col == self_col) | (col >= K_STORE)
    s = jnp.where(mask, _NEG, s)
    s_ref[...] = s
    gm_ref[...] = jnp.max(s.reshape(B, TILE_N // 128, 128), axis=2).transpose(1, 0)[None]


@jax.jit
def _scores(q, keys_pad, x_idx):
    return pl.pallas_call(
        _score_body,
        grid=(N_TILES,),
        in_specs=[
            pl.BlockSpec((B, D), lambda j: (0, 0)),
            pl.BlockSpec((TILE_N, D), lambda j: (j, 0)),
            pl.BlockSpec((B, 1), lambda j: (0, 0)),
        ],
        out_specs=[
            pl.BlockSpec((B, TILE_N), lambda j: (0, j)),
            pl.BlockSpec((1, TILE_N // 128, B), lambda j: (j, 0, 0)),
        ],
        out_shape=[
            jax.ShapeDtypeStruct((B, N_PAD), jnp.float32),
            jax.ShapeDtypeStruct((N_TILES, TILE_N // 128, B), jnp.float32),
        ],
    )(q, keys_pad, x_idx.astype(jnp.int32).reshape(B, 1))


def _merge32(ka, va, kb, vb):
    rb0 = lax.rev(kb[0], (0,))
    rb1 = lax.rev(kb[1], (0,))
    rv0 = lax.rev(vb[0], (0,))
    rv1 = lax.rev(vb[1], (0,))
    ge0 = ka[0] >= rb1
    k0 = jnp.where(ge0, ka[0], rb1)
    v0 = jnp.where(ge0, va[0], rv1)
    ge1 = ka[1] >= rb0
    k1 = jnp.where(ge1, ka[1], rb0)
    v1 = jnp.where(ge1, va[1], rv0)
    geh = k0 >= k1
    hk = jnp.where(geh, k0, k1)
    hv = jnp.where(geh, v0, v1)
    lk = jnp.where(geh, k1, k0)
    lv = jnp.where(geh, v1, v0)
    hk, hv = plsc.sort_key_val(hk, hv, descending=True)
    lk, lv = plsc.sort_key_val(lk, lv, descending=True)
    return (hk, lk), (hv, lv)


def _fold_chunk(acc, k16, v16):
    neg = jnp.full((16,), _NEG, jnp.float32)
    zero = jnp.zeros((16,), jnp.int32)
    sk, sv = plsc.sort_key_val(k16, v16, descending=True)
    if acc is None:
        return (sk, neg), (sv, zero)
    ka, va = acc
    return _merge32(ka, va, (sk, neg), (sv, zero))


def _select_body(gm_hbm, scores_hbm, keys_hbm, nf_hbm, sv_hbm,
                 labels_hbm, knnk_hbm, nfg_hbm,
                 gm_v, cand_v, idx_v, gids_v, knns_v, krows_v, nfrows_v,
                 labels_v, sem):
    nc = 2
    wid = lax.axis_index("s") * nc + lax.axis_index("c")
    iota = lax.iota(jnp.int32, 16)

    def row_body(t, carry):
        r = wid * 4 + t
        pltpu.sync_copy(gm_hbm.at[r], gm_v)

        acc = None
        for j in range(NCHUNK):
            k16 = gm_v[pl.ds(16 * j, 16)]
            v16 = iota + 16 * j
            acc = _fold_chunk(acc, k16, v16)
        gk, gv = acc
        gids_v[pl.ds(0, 16)] = gv[0]
        gids_v[pl.ds(16, 16)] = gv[1]
        idx_v[pl.ds(0, 16)] = gv[0] + r * G
        idx_v[pl.ds(16, 16)] = gv[1] + r * G

        pltpu.async_copy(scores_hbm.at[idx_v], cand_v, sem).wait()

        submax = []
        for g in range(32):
            m = cand_v[g, pl.ds(0, 16)]
            for i in range(1, 8):
                m = jnp.maximum(m, cand_v[g, pl.ds(16 * i, 16)])
            submax.append(m)

        acc = None
        for g in range(32):
            acc = _fold_chunk(acc, submax[g], iota + 16 * g)
        pk, pv = acc

        acc = None
        for half in range(2):
            p = pv[half]
            g = lax.shift_right_logical(p, 4)
            j = jnp.bitwise_and(p, 15)
            gidv = plsc.load_gather(gids_v, [g])
            base = gidv * 128 + j
            for i in range(8):
                k16 = plsc.load_gather(cand_v, [g, j + 16 * i])
                acc = _fold_chunk(acc, k16, base + 16 * i)
        fk, fv = acc
        knns_v[pl.ds(0, 16)] = fv[0]
        knns_v[pl.ds(16, 16)] = fv[1]

        ck = pltpu.async_copy(keys_hbm.at[knns_v], krows_v, sem)
        cn = pltpu.async_copy(nf_hbm.at[knns_v], nfrows_v, sem)
        cl = pltpu.async_copy(sv_hbm.at[knns_v], labels_v, sem)
        ck.wait()
        cn.wait()
        cl.wait()
        pltpu.sync_copy(krows_v, knnk_hbm.at[r])
        pltpu.sync_copy(nfrows_v, nfg_hbm.at[r])
        pltpu.sync_copy(labels_v, labels_hbm.at[r])
        return carry

    lax.fori_loop(0, 4, row_body, 0)


@jax.jit
def _select(gm1, scores_resh, keys_store, neighbor_feats, store_vals):
    mesh = plsc.VectorSubcoreMesh(core_axis_name="c", subcore_axis_name="s")
    return pl.kernel(
        _select_body,
        mesh=mesh,
        out_type=[
            jax.ShapeDtypeStruct((B, K), jnp.int32),
            jax.ShapeDtypeStruct((B, K, D), jnp.float32),
            jax.ShapeDtypeStruct((B, K, D), jnp.float32),
        ],
        scratch_types=[
            pltpu.VMEM((G,), jnp.float32),
            pltpu.VMEM((32, 128), jnp.float32),
            pltpu.VMEM((32,), jnp.int32),
            pltpu.VMEM((32,), jnp.int32),
            pltpu.VMEM((32,), jnp.int32),
            pltpu.VMEM((K, D), jnp.float32),
            pltpu.VMEM((K, D), jnp.float32),
            pltpu.VMEM((K,), jnp.int32),
            pltpu.SemaphoreType.DMA,
        ],
        compiler_params=pltpu.CompilerParams(needs_layout_passes=False),
    )(gm1, scores_resh, keys_store, neighbor_feats, store_vals)


ENC_ROWS = 32


def _encode_body(x_ref, m_ref, wenc_ref, benc_ref, wcls_ref, bcls_ref,
                 t_ref, p_ref):
    xm = x_ref[...] * m_ref[...][:, :, None]
    pooled = jnp.sum(xm, axis=1) / jnp.maximum(
        jnp.sum(m_ref[...], axis=1), 1.0)[:, None]
    t = jnp.tanh(jax.lax.dot_general(pooled, wenc_ref[...],
                                     (((1,), (0,)), ((), ())),
                                     preferred_element_type=jnp.float32)
                 + benc_ref[...])
    logits = jax.lax.dot_general(t, wcls_ref[...], (((1,), (0,)), ((), ())),
                                 preferred_element_type=jnp.float32) + bcls_ref[...]
    mx = jnp.max(logits, axis=1, keepdims=True)
    e = jnp.exp(logits - mx)
    t_ref[...] = t
    p_ref[...] = e / jnp.sum(e, axis=1, keepdims=True)


@jax.jit
def _encode(x, x_mask, W_enc, b_enc, W_cls, b_cls):
    return pl.pallas_call(
        _encode_body,
        grid=(B // ENC_ROWS,),
        in_specs=[
            pl.BlockSpec((ENC_ROWS, S, D), lambda i: (i, 0, 0)),
            pl.BlockSpec((ENC_ROWS, S), lambda i: (i, 0)),
            pl.BlockSpec((D, D), lambda i: (0, 0)),
            pl.BlockSpec((1, D), lambda i: (0, 0)),
            pl.BlockSpec((D, NUM_CLASSES), lambda i: (0, 0)),
            pl.BlockSpec((1, NUM_CLASSES), lambda i: (0, 0)),
        ],
        out_specs=[
            pl.BlockSpec((ENC_ROWS, D), lambda i: (i, 0)),
            pl.BlockSpec((ENC_ROWS, NUM_CLASSES), lambda i: (i, 0)),
        ],
        out_shape=[
            jax.ShapeDtypeStruct((B, D), jnp.float32),
            jax.ShapeDtypeStruct((B, NUM_CLASSES), jnp.float32),
        ],
    )(x, x_mask, W_enc, b_enc.reshape(1, D), W_cls, b_cls.reshape(1, NUM_CLASSES))


TAIL_ROWS = 32


def _tail_body(t_ref, lab_ref, knnk_ref, nfg_ref, mp_ref,
               wenc_ref, benc_ref, w1_ref, b1_ref, w2_ref, b2_ref, out_ref):
    t = t_ref[...]
    dists = jnp.sum((t[:, None, :] - knnk_ref[...]) ** 2, axis=-1)
    u = -dists / TEMP
    um = jnp.max(u, axis=1, keepdims=True)
    e = jnp.exp(u - um)
    pr = e / jnp.sum(e, axis=1, keepdims=True)

    nf = nfg_ref[...].reshape(TAIL_ROWS * K, D)
    nb = jnp.tanh(jax.lax.dot_general(nf, wenc_ref[...],
                                      (((1,), (0,)), ((), ())),
                                      preferred_element_type=jnp.float32)
                  + benc_ref[...])
    nrep = jnp.sum(pr[:, :, None] * nb.reshape(TAIL_ROWS, K, D), axis=1)
    h = jnp.concatenate([t, nrep], axis=1)
    h = jax.lax.dot_general(h, w1_ref[...], (((1,), (0,)), ((), ())),
                            preferred_element_type=jnp.float32) + b1_ref[...]
    pk = jax.nn.sigmoid(
        jax.lax.dot_general(h, w2_ref[...], (((1,), (0,)), ((), ())),
                            preferred_element_type=jnp.float32) + b2_ref[...])
    lab = lab_ref[...]
    cls_iota = jax.lax.broadcasted_iota(jnp.int32, (TAIL_ROWS, NUM_CLASSES), 1)
    acc = jnp.zeros((TAIL_ROWS, NUM_CLASSES), jnp.float32)
    for k in range(K):
        acc = acc + jnp.where(lab[:, k:k + 1] == cls_iota, pr[:, k:k + 1], 0.0)
    out_ref[...] = jnp.log(pk * acc + (1.0 - pk) * mp_ref[...] + 1e-12)


@jax.jit
def _tail(text_rep, labels, knnk, nf_g, model_prob, W_enc, b_enc, W1, b1, W2, b2):
    return pl.pallas_call(
        _tail_body,
        grid=(B // TAIL_ROWS,),
        in_specs=[
            pl.BlockSpec((TAIL_ROWS, D), lambda i: (i, 0)),
            pl.BlockSpec((TAIL_ROWS, K), lambda i: (i, 0)),
            pl.BlockSpec((TAIL_ROWS, K, D), lambda i: (i, 0, 0)),
            pl.BlockSpec((TAIL_ROWS, K, D), lambda i: (i, 0, 0)),
            pl.BlockSpec((TAIL_ROWS, NUM_CLASSES), lambda i: (i, 0)),
            pl.BlockSpec((D, D), lambda i: (0, 0)),
            pl.BlockSpec((1, D), lambda i: (0, 0)),
            pl.BlockSpec((2 * D, 2 * D), lambda i: (0, 0)),
            pl.BlockSpec((1, 2 * D), lambda i: (0, 0)),
            pl.BlockSpec((2 * D, 1), lambda i: (0, 0)),
            pl.BlockSpec((1, 1), lambda i: (0, 0)),
        ],
        out_specs=pl.BlockSpec((TAIL_ROWS, NUM_CLASSES), lambda i: (i, 0)),
        out_shape=jax.ShapeDtypeStruct((B, NUM_CLASSES), jnp.float32),
    )(text_rep, labels, knnk, nf_g, model_prob,
      W_enc, b_enc.reshape(1, D), W1, b1.reshape(1, 2 * D), W2,
      b2.reshape(1, 1))


def kernel(x, x_mask, x_idx, keys_store, store_vals, neighbor_feats,
           W_enc, b_enc, W_cls, b_cls, W1, b1, W2, b2):
    q = keys_store[x_idx]
    scores, gm_t = _scores(q, keys_store, x_idx)
    gm1 = gm_t.transpose(2, 0, 1).reshape(B, G)

    labels, knnk, nf_g = _select(gm1, scores.reshape(B * G, 128),
                                 keys_store, neighbor_feats, store_vals)

    text_rep, model_prob = _encode(x, x_mask, W_enc, b_enc, W_cls, b_cls)

    return _tail(text_rep, labels, knnk, nf_g, model_prob,
                 W_enc, b_enc, W1, b1, W2, b2)


# speedup vs baseline: 32.0972x; 1.0033x over previous
import functools

import jax
import jax.numpy as jnp
from jax import lax
from jax.experimental import pallas as pl
from jax.experimental.pallas import tpu as pltpu, tpu_sc as plsc

B = 128
S = 128
D = 768
K_STORE = 100000
NUM_CLASSES = 1000
K = 32
TEMP = 10.0

TILE_N = 5120
N_PAD = 102400
N_TILES = N_PAD // TILE_N
G = N_PAD // 128
NCHUNK = G // 16

_NEG = -3.0e38


def _score_body(q_ref, k_ref, xidx_ref, s_ref, gm_ref):
    j = pl.program_id(0)
    q = q_ref[...]
    kb = k_ref[...]
    dot = jax.lax.dot_general(q, kb, (((1,), (1,)), ((), ())),
                              preferred_element_type=jnp.float32)
    k2 = jnp.sum(kb * kb, axis=1)
    s = 2.0 * dot - k2[None, :]
    col = jax.lax.broadcasted_iota(jnp.int32, (B, TILE_N), 1) + j * TILE_N
    self_col = xidx_ref[...]
    mask = (col == self_col) | (col >= K_STORE)
    s = jnp.where(mask, _NEG, s)
    s_ref[...] = s
    gm_ref[...] = jnp.max(s.reshape(B, TILE_N // 128, 128), axis=2).transpose(1, 0)[None]


@jax.jit
def _scores(q, keys_pad, x_idx):
    return pl.pallas_call(
        _score_body,
        grid=(N_TILES,),
        in_specs=[
            pl.BlockSpec((B, D), lambda j: (0, 0)),
            pl.BlockSpec((TILE_N, D), lambda j: (j, 0)),
            pl.BlockSpec((B, 1), lambda j: (0, 0)),
        ],
        out_specs=[
            pl.BlockSpec((B, TILE_N), lambda j: (0, j)),
            pl.BlockSpec((1, TILE_N // 128, B), lambda j: (j, 0, 0)),
        ],
        out_shape=[
            jax.ShapeDtypeStruct((B, N_PAD), jnp.float32),
            jax.ShapeDtypeStruct((N_TILES, TILE_N // 128, B), jnp.float32),
        ],
    )(q, keys_pad, x_idx.astype(jnp.int32).reshape(B, 1))


def _merge32(ka, va, kb, vb):
    rb0 = lax.rev(kb[0], (0,))
    rb1 = lax.rev(kb[1], (0,))
    rv0 = lax.rev(vb[0], (0,))
    rv1 = lax.rev(vb[1], (0,))
    ge0 = ka[0] >= rb1
    k0 = jnp.where(ge0, ka[0], rb1)
    v0 = jnp.where(ge0, va[0], rv1)
    ge1 = ka[1] >= rb0
    k1 = jnp.where(ge1, ka[1], rb0)
    v1 = jnp.where(ge1, va[1], rv0)
    geh = k0 >= k1
    hk = jnp.where(geh, k0, k1)
    hv = jnp.where(geh, v0, v1)
    lk = jnp.where(geh, k1, k0)
    lv = jnp.where(geh, v1, v0)
    hk, hv = plsc.sort_key_val(hk, hv, descending=True)
    lk, lv = plsc.sort_key_val(lk, lv, descending=True)
    return (hk, lk), (hv, lv)


def _fold_chunk(acc, k16, v16):
    neg = jnp.full((16,), _NEG, jnp.float32)
    zero = jnp.zeros((16,), jnp.int32)
    sk, sv = plsc.sort_key_val(k16, v16, descending=True)
    if acc is None:
        return (sk, neg), (sv, zero)
    ka, va = acc
    return _merge32(ka, va, (sk, neg), (sv, zero))


def _select_body(gm_hbm, scores_hbm, keys_hbm, nf_hbm, sv_hbm,
                 labels_hbm, knnk_hbm, nfg_hbm,
                 gm_v, cand_v, idx_v, gids_v, knns_v, krows_v, nfrows_v,
                 labels_v, sem):
    nc = 2
    wid = lax.axis_index("s") * nc + lax.axis_index("c")
    iota = lax.iota(jnp.int32, 16)

    def row_body(t, carry):
        r = wid * 4 + t
        pltpu.sync_copy(gm_hbm.at[r], gm_v)

        acc = None
        for j in range(NCHUNK):
            k16 = gm_v[pl.ds(16 * j, 16)]
            v16 = iota + 16 * j
            acc = _fold_chunk(acc, k16, v16)
        gk, gv = acc
        gids_v[pl.ds(0, 16)] = gv[0]
        gids_v[pl.ds(16, 16)] = gv[1]
        idx_v[pl.ds(0, 16)] = gv[0] + r * G
        idx_v[pl.ds(16, 16)] = gv[1] + r * G

        pltpu.async_copy(scores_hbm.at[idx_v], cand_v, sem).wait()

        submax = []
        for g in range(32):
            m = cand_v[g, pl.ds(0, 16)]
            for i in range(1, 8):
                m = jnp.maximum(m, cand_v[g, pl.ds(16 * i, 16)])
            submax.append(m)

        acc = None
        for g in range(32):
            acc = _fold_chunk(acc, submax[g], iota + 16 * g)
        pk, pv = acc

        acc = None
        for half in range(2):
            p = pv[half]
            g = lax.shift_right_logical(p, 4)
            j = jnp.bitwise_and(p, 15)
            gidv = plsc.load_gather(gids_v, [g])
            base = gidv * 128 + j
            for i in range(8):
                k16 = plsc.load_gather(cand_v, [g, j + 16 * i])
                acc = _fold_chunk(acc, k16, base + 16 * i)
        fk, fv = acc
        knns_v[pl.ds(0, 16)] = fv[0]
        knns_v[pl.ds(16, 16)] = fv[1]

        ck = pltpu.async_copy(keys_hbm.at[knns_v], krows_v, sem)
        cn = pltpu.async_copy(nf_hbm.at[knns_v], nfrows_v, sem)
        cl = pltpu.async_copy(sv_hbm.at[knns_v], labels_v, sem)
        ck.wait()
        cn.wait()
        cl.wait()
        pltpu.sync_copy(krows_v, knnk_hbm.at[r])
        pltpu.sync_copy(nfrows_v, nfg_hbm.at[r])
        pltpu.sync_copy(labels_v, labels_hbm.at[r])
        return carry

    lax.fori_loop(0, 4, row_body, 0)


@jax.jit
def _select(gm1, scores_resh, keys_store, neighbor_feats, store_vals):
    mesh = plsc.VectorSubcoreMesh(core_axis_name="c", subcore_axis_name="s")
    return pl.kernel(
        _select_body,
        mesh=mesh,
        out_type=[
            jax.ShapeDtypeStruct((B, K), jnp.int32),
            jax.ShapeDtypeStruct((B, K, D), jnp.float32),
            jax.ShapeDtypeStruct((B, K, D), jnp.float32),
        ],
        scratch_types=[
            pltpu.VMEM((G,), jnp.float32),
            pltpu.VMEM((32, 128), jnp.float32),
            pltpu.VMEM((32,), jnp.int32),
            pltpu.VMEM((32,), jnp.int32),
            pltpu.VMEM((32,), jnp.int32),
            pltpu.VMEM((K, D), jnp.float32),
            pltpu.VMEM((K, D), jnp.float32),
            pltpu.VMEM((K,), jnp.int32),
            pltpu.SemaphoreType.DMA,
        ],
        compiler_params=pltpu.CompilerParams(needs_layout_passes=False),
    )(gm1, scores_resh, keys_store, neighbor_feats, store_vals)


ENC_ROWS = 32


def _encode_body(x_ref, m_ref, wenc_ref, benc_ref, wcls_ref, bcls_ref,
                 t_ref, p_ref):
    xm = x_ref[...] * m_ref[...][:, :, None]
    pooled = jnp.sum(xm, axis=1) / jnp.maximum(
        jnp.sum(m_ref[...], axis=1), 1.0)[:, None]
    t = jnp.tanh(jax.lax.dot_general(pooled, wenc_ref[...],
                                     (((1,), (0,)), ((), ())),
                                     preferred_element_type=jnp.float32)
                 + benc_ref[...])
    logits = jax.lax.dot_general(t, wcls_ref[...], (((1,), (0,)), ((), ())),
                                 preferred_element_type=jnp.float32) + bcls_ref[...]
    mx = jnp.max(logits, axis=1, keepdims=True)
    e = jnp.exp(logits - mx)
    t_ref[...] = t
    p_ref[...] = e / jnp.sum(e, axis=1, keepdims=True)


@jax.jit
def _encode(x, x_mask, W_enc, b_enc, W_cls, b_cls):
    return pl.pallas_call(
        _encode_body,
        grid=(B // ENC_ROWS,),
        in_specs=[
            pl.BlockSpec((ENC_ROWS, S, D), lambda i: (i, 0, 0)),
            pl.BlockSpec((ENC_ROWS, S), lambda i: (i, 0)),
            pl.BlockSpec((D, D), lambda i: (0, 0)),
            pl.BlockSpec((1, D), lambda i: (0, 0)),
            pl.BlockSpec((D, NUM_CLASSES), lambda i: (0, 0)),
            pl.BlockSpec((1, NUM_CLASSES), lambda i: (0, 0)),
        ],
        out_specs=[
            pl.BlockSpec((ENC_ROWS, D), lambda i: (i, 0)),
            pl.BlockSpec((ENC_ROWS, NUM_CLASSES), lambda i: (i, 0)),
        ],
        out_shape=[
            jax.ShapeDtypeStruct((B, D), jnp.float32),
            jax.ShapeDtypeStruct((B, NUM_CLASSES), jnp.float32),
        ],
    )(x, x_mask, W_enc, b_enc.reshape(1, D), W_cls, b_cls.reshape(1, NUM_CLASSES))


TAIL_ROWS = 32


def _tail_body(t_ref, lab_ref, knnk_ref, nfg_ref, mp_ref,
               wenc_ref, benc_ref, w1_ref, b1_ref, w2_ref, b2_ref, out_ref):
    t = t_ref[...]
    dists = jnp.sum((t[:, None, :] - knnk_ref[...]) ** 2, axis=-1)
    u = -dists / TEMP
    um = jnp.max(u, axis=1, keepdims=True)
    e = jnp.exp(u - um)
    pr = e / jnp.sum(e, axis=1, keepdims=True)

    nf = nfg_ref[...].reshape(TAIL_ROWS * K, D)
    nb = jnp.tanh(jax.lax.dot_general(nf, wenc_ref[...],
                                      (((1,), (0,)), ((), ())),
                                      preferred_element_type=jnp.float32)
                  + benc_ref[...])
    nrep = jnp.sum(pr[:, :, None] * nb.reshape(TAIL_ROWS, K, D), axis=1)
    h = jnp.concatenate([t, nrep], axis=1)
    h = jax.lax.dot_general(h, w1_ref[...], (((1,), (0,)), ((), ())),
                            preferred_element_type=jnp.float32) + b1_ref[...]
    pk = jax.nn.sigmoid(
        jax.lax.dot_general(h, w2_ref[...], (((1,), (0,)), ((), ())),
                            preferred_element_type=jnp.float32) + b2_ref[...])
    lab = lab_ref[...]
    cls_iota = jax.lax.broadcasted_iota(jnp.int32, (TAIL_ROWS, NUM_CLASSES), 1)
    acc = jnp.zeros((TAIL_ROWS, NUM_CLASSES), jnp.float32)
    for k in range(K):
        acc = acc + jnp.where(lab[:, k:k + 1] == cls_iota, pr[:, k:k + 1], 0.0)
    out_ref[...] = jnp.log(pk * acc + (1.0 - pk) * mp_ref[...] + 1e-12)


@jax.jit
def _tail(text_rep, labels, knnk, nf_g, model_prob, W_enc, b_enc, W1, b1, W2, b2):
    return pl.pallas_call(
        _tail_body,
        grid=(B // TAIL_ROWS,),
        in_specs=[
            pl.BlockSpec((TAIL_ROWS, D), lambda i: (i, 0)),
            pl.BlockSpec((TAIL_ROWS, K), lambda i: (i, 0)),
            pl.BlockSpec((TAIL_ROWS, K, D), lambda i: (i, 0, 0)),
            pl.BlockSpec((TAIL_ROWS, K, D), lambda i: (i, 0, 0)),
            pl.BlockSpec((TAIL_ROWS, NUM_CLASSES), lambda i: (i, 0)),
            pl.BlockSpec((D, D), lambda i: (0, 0)),
            pl.BlockSpec((1, D), lambda i: (0, 0)),
            pl.BlockSpec((2 * D, 2 * D), lambda i: (0, 0)),
            pl.BlockSpec((1, 2 * D), lambda i: (0, 0)),
            pl.BlockSpec((2 * D, 1), lambda i: (0, 0)),
            pl.BlockSpec((1, 1), lambda i: (0, 0)),
        ],
        out_specs=pl.BlockSpec((TAIL_ROWS, NUM_CLASSES), lambda i: (i, 0)),
        out_shape=jax.ShapeDtypeStruct((B, NUM_CLASSES), jnp.float32),
    )(text_rep, labels, knnk, nf_g, model_prob,
      W_enc, b_enc.reshape(1, D), W1, b1.reshape(1, 2 * D), W2,
      b2.reshape(1, 1))


def kernel(x, x_mask, x_idx, keys_store, store_vals, neighbor_feats,
           W_enc, b_enc, W_cls, b_cls, W1, b1, W2, b2):
    q = keys_store[x_idx]
    scores, gm_t = _scores(q, keys_store, x_idx)
    gm1 = gm_t.transpose(2, 0, 1).reshape(B, G)

    labels, knnk, nf_g = _select(gm1, scores.reshape(B * G, 128),
                                 keys_store, neighbor_feats, store_vals)

    text_rep, model_prob = _encode(x, x_mask, W_enc, b_enc, W_cls, b_cls)

    return _tail(text_rep, labels, knnk, nf_g, model_prob,
                 W_enc, b_enc, W1, b1, W2, b2)


# speedup vs baseline: 32.4122x; 1.0098x over previous
import functools

import jax
import jax.numpy as jnp
from jax import lax
from jax.experimental import pallas as pl
from jax.experimental.pallas import tpu as pltpu, tpu_sc as plsc

B = 128
S = 128
D = 768
K_STORE = 100000
NUM_CLASSES = 1000
K = 32
TEMP = 10.0

TILE_N = 5120
N_PAD = 102400
N_TILES = N_PAD // TILE_N
G = N_PAD // 128
NCHUNK = G // 16

_NEG = -3.0e38


def _score_body(q_ref, k_ref, xidx_ref, s_ref, gm_ref):
    j = pl.program_id(0)
    q = q_ref[...]
    kb = k_ref[...]
    dot = jax.lax.dot_general(q, kb, (((1,), (1,)), ((), ())),
                              preferred_element_type=jnp.float32)
    k2 = jnp.sum(kb * kb, axis=1)
    s = 2.0 * dot - k2[None, :]
    col = jax.lax.broadcasted_iota(jnp.int32, (B, TILE_N), 1) + j * TILE_N
    self_col = xidx_ref[...]
    mask = (col == self_col) | (col >= K_STORE)
    s = jnp.where(mask, _NEG, s)
    s_ref[...] = s
    gm_ref[...] = jnp.max(s.reshape(B, TILE_N // 128, 128), axis=2).transpose(1, 0)[None]


@jax.jit
def _scores(q, keys_pad, x_idx):
    return pl.pallas_call(
        _score_body,
        grid=(N_TILES,),
        in_specs=[
            pl.BlockSpec((B, D), lambda j: (0, 0)),
            pl.BlockSpec((TILE_N, D), lambda j: (j, 0)),
            pl.BlockSpec((B, 1), lambda j: (0, 0)),
        ],
        out_specs=[
            pl.BlockSpec((B, TILE_N), lambda j: (0, j)),
            pl.BlockSpec((1, TILE_N // 128, B), lambda j: (j, 0, 0)),
        ],
        out_shape=[
            jax.ShapeDtypeStruct((B, N_PAD), jnp.float32),
            jax.ShapeDtypeStruct((N_TILES, TILE_N // 128, B), jnp.float32),
        ],
    )(q, keys_pad, x_idx.astype(jnp.int32).reshape(B, 1))


def _merge32(ka, va, kb, vb):
    rb0 = lax.rev(kb[0], (0,))
    rb1 = lax.rev(kb[1], (0,))
    rv0 = lax.rev(vb[0], (0,))
    rv1 = lax.rev(vb[1], (0,))
    ge0 = ka[0] >= rb1
    k0 = jnp.where(ge0, ka[0], rb1)
    v0 = jnp.where(ge0, va[0], rv1)
    ge1 = ka[1] >= rb0
    k1 = jnp.where(ge1, ka[1], rb0)
    v1 = jnp.where(ge1, va[1], rv0)
    geh = k0 >= k1
    hk = jnp.where(geh, k0, k1)
    hv = jnp.where(geh, v0, v1)
    lk = jnp.where(geh, k1, k0)
    lv = jnp.where(geh, v1, v0)
    hk, hv = plsc.sort_key_val(hk, hv, descending=True)
    lk, lv = plsc.sort_key_val(lk, lv, descending=True)
    return (hk, lk), (hv, lv)


def _fold_chunk(acc, k16, v16):
    neg = jnp.full((16,), _NEG, jnp.float32)
    zero = jnp.zeros((16,), jnp.int32)
    sk, sv = plsc.sort_key_val(k16, v16, descending=True)
    if acc is None:
        return (sk, neg), (sv, zero)
    ka, va = acc
    return _merge32(ka, va, (sk, neg), (sv, zero))


def _select_body(gm_hbm, scores_hbm, keys_hbm, nf_hbm, sv_hbm,
                 labels_hbm, comb_hbm,
                 gm_v, cand_v, idx_v, gids_v, knns_v, comb_v,
                 labels_v, sem, semw):
    nc = 2
    wid = lax.axis_index("s") * nc + lax.axis_index("c")
    iota = lax.iota(jnp.int32, 16)
    pltpu.sync_copy(gm_hbm.at[pl.ds(wid * 4, 4)], gm_v)

    def row_body(t, carry):
        r = wid * 4 + t

        acc = None
        for j in range(NCHUNK):
            k16 = gm_v[t, pl.ds(16 * j, 16)]
            v16 = iota + 16 * j
            acc = _fold_chunk(acc, k16, v16)
        gk, gv = acc
        gids_v[pl.ds(0, 16)] = gv[0]
        gids_v[pl.ds(16, 16)] = gv[1]
        idx_v[pl.ds(0, 16)] = gv[0] + r * G
        idx_v[pl.ds(16, 16)] = gv[1] + r * G

        pltpu.async_copy(scores_hbm.at[idx_v], cand_v, sem).wait()

        submax = []
        for g in range(32):
            m = cand_v[g, pl.ds(0, 16)]
            for i in range(1, 8):
                m = jnp.maximum(m, cand_v[g, pl.ds(16 * i, 16)])
            submax.append(m)

        acc = None
        for g in range(32):
            acc = _fold_chunk(acc, submax[g], iota + 16 * g)
        pk, pv = acc

        acc = None
        for half in range(2):
            p = pv[half]
            g = lax.shift_right_logical(p, 4)
            j = jnp.bitwise_and(p, 15)
            gidv = plsc.load_gather(gids_v, [g])
            base = gidv * 128 + j
            for i in range(8):
                k16 = plsc.load_gather(cand_v, [g, j + 16 * i])
                acc = _fold_chunk(acc, k16, base + 16 * i)
        fk, fv = acc
        knns_v[pl.ds(0, 16)] = fv[0]
        knns_v[pl.ds(16, 16)] = fv[1]

        @pl.when(t > 0)
        def _():
            pltpu.make_async_copy(comb_v, comb_hbm.at[r - 1], semw).wait()
            pltpu.make_async_copy(labels_v, labels_hbm.at[r - 1], semw).wait()

        ck = pltpu.async_copy(keys_hbm.at[knns_v], comb_v.at[pl.ds(0, K)], sem)
        cn = pltpu.async_copy(nf_hbm.at[knns_v], comb_v.at[pl.ds(K, K)], sem)
        cl = pltpu.async_copy(sv_hbm.at[knns_v], labels_v, sem)
        ck.wait()
        cn.wait()
        cl.wait()
        pltpu.async_copy(comb_v, comb_hbm.at[r], semw)
        pltpu.async_copy(labels_v, labels_hbm.at[r], semw)
        return carry

    lax.fori_loop(0, 4, row_body, 0)
    pltpu.make_async_copy(comb_v, comb_hbm.at[wid * 4 + 3], semw).wait()
    pltpu.make_async_copy(labels_v, labels_hbm.at[wid * 4 + 3], semw).wait()


@jax.jit
def _select(gm1, scores_resh, keys_store, neighbor_feats, store_vals):
    mesh = plsc.VectorSubcoreMesh(core_axis_name="c", subcore_axis_name="s")
    return pl.kernel(
        _select_body,
        mesh=mesh,
        out_type=[
            jax.ShapeDtypeStruct((B, K), jnp.int32),
            jax.ShapeDtypeStruct((B, 2 * K, D), jnp.float32),
        ],
        scratch_types=[
            pltpu.VMEM((4, G), jnp.float32),
            pltpu.VMEM((32, 128), jnp.float32),
            pltpu.VMEM((32,), jnp.int32),
            pltpu.VMEM((32,), jnp.int32),
            pltpu.VMEM((32,), jnp.int32),
            pltpu.VMEM((2 * K, D), jnp.float32),
            pltpu.VMEM((K,), jnp.int32),
            pltpu.SemaphoreType.DMA,
            pltpu.SemaphoreType.DMA,
        ],
        compiler_params=pltpu.CompilerParams(needs_layout_passes=False),
    )(gm1, scores_resh, keys_store, neighbor_feats, store_vals)


ENC_ROWS = 32


def _encode_body(x_ref, m_ref, wenc_ref, benc_ref, wcls_ref, bcls_ref,
                 t_ref, p_ref):
    xm = x_ref[...] * m_ref[...][:, :, None]
    pooled = jnp.sum(xm, axis=1) / jnp.maximum(
        jnp.sum(m_ref[...], axis=1), 1.0)[:, None]
    t = jnp.tanh(jax.lax.dot_general(pooled, wenc_ref[...],
                                     (((1,), (0,)), ((), ())),
                                     preferred_element_type=jnp.float32)
                 + benc_ref[...])
    logits = jax.lax.dot_general(t, wcls_ref[...], (((1,), (0,)), ((), ())),
                                 preferred_element_type=jnp.float32) + bcls_ref[...]
    mx = jnp.max(logits, axis=1, keepdims=True)
    e = jnp.exp(logits - mx)
    t_ref[...] = t
    p_ref[...] = e / jnp.sum(e, axis=1, keepdims=True)


@jax.jit
def _encode(x, x_mask, W_enc, b_enc, W_cls, b_cls):
    return pl.pallas_call(
        _encode_body,
        grid=(B // ENC_ROWS,),
        in_specs=[
            pl.BlockSpec((ENC_ROWS, S, D), lambda i: (i, 0, 0)),
            pl.BlockSpec((ENC_ROWS, S), lambda i: (i, 0)),
            pl.BlockSpec((D, D), lambda i: (0, 0)),
            pl.BlockSpec((1, D), lambda i: (0, 0)),
            pl.BlockSpec((D, NUM_CLASSES), lambda i: (0, 0)),
            pl.BlockSpec((1, NUM_CLASSES), lambda i: (0, 0)),
        ],
        out_specs=[
            pl.BlockSpec((ENC_ROWS, D), lambda i: (i, 0)),
            pl.BlockSpec((ENC_ROWS, NUM_CLASSES), lambda i: (i, 0)),
        ],
        out_shape=[
            jax.ShapeDtypeStruct((B, D), jnp.float32),
            jax.ShapeDtypeStruct((B, NUM_CLASSES), jnp.float32),
        ],
    )(x, x_mask, W_enc, b_enc.reshape(1, D), W_cls, b_cls.reshape(1, NUM_CLASSES))


TAIL_ROWS = 32


def _tail_body(t_ref, lab_ref, comb_ref, mp_ref,
               wenc_ref, benc_ref, w1_ref, b1_ref, w2_ref, b2_ref, out_ref):
    t = t_ref[...]
    knnk = comb_ref[:, :K, :]
    dists = jnp.sum((t[:, None, :] - knnk) ** 2, axis=-1)
    u = -dists / TEMP
    um = jnp.max(u, axis=1, keepdims=True)
    e = jnp.exp(u - um)
    pr = e / jnp.sum(e, axis=1, keepdims=True)

    nf = comb_ref[:, K:, :].reshape(TAIL_ROWS * K, D)
    nb = jnp.tanh(jax.lax.dot_general(nf, wenc_ref[...],
                                      (((1,), (0,)), ((), ())),
                                      preferred_element_type=jnp.float32)
                  + benc_ref[...])
    nrep = jnp.sum(pr[:, :, None] * nb.reshape(TAIL_ROWS, K, D), axis=1)
    h = jnp.concatenate([t, nrep], axis=1)
    h = jax.lax.dot_general(h, w1_ref[...], (((1,), (0,)), ((), ())),
                            preferred_element_type=jnp.float32) + b1_ref[...]
    pk = jax.nn.sigmoid(
        jax.lax.dot_general(h, w2_ref[...], (((1,), (0,)), ((), ())),
                            preferred_element_type=jnp.float32) + b2_ref[...])
    lab = lab_ref[...]
    cls_iota = jax.lax.broadcasted_iota(jnp.int32, (TAIL_ROWS, NUM_CLASSES), 1)
    acc = jnp.zeros((TAIL_ROWS, NUM_CLASSES), jnp.float32)
    for k in range(K):
        acc = acc + jnp.where(lab[:, k:k + 1] == cls_iota, pr[:, k:k + 1], 0.0)
    out_ref[...] = jnp.log(pk * acc + (1.0 - pk) * mp_ref[...] + 1e-12)


@jax.jit
def _tail(text_rep, labels, comb, model_prob, W_enc, b_enc, W1, b1, W2, b2):
    return pl.pallas_call(
        _tail_body,
        grid=(B // TAIL_ROWS,),
        in_specs=[
            pl.BlockSpec((TAIL_ROWS, D), lambda i: (i, 0)),
            pl.BlockSpec((TAIL_ROWS, K), lambda i: (i, 0)),
            pl.BlockSpec((TAIL_ROWS, 2 * K, D), lambda i: (i, 0, 0)),
            pl.BlockSpec((TAIL_ROWS, NUM_CLASSES), lambda i: (i, 0)),
            pl.BlockSpec((D, D), lambda i: (0, 0)),
            pl.BlockSpec((1, D), lambda i: (0, 0)),
            pl.BlockSpec((2 * D, 2 * D), lambda i: (0, 0)),
            pl.BlockSpec((1, 2 * D), lambda i: (0, 0)),
            pl.BlockSpec((2 * D, 1), lambda i: (0, 0)),
            pl.BlockSpec((1, 1), lambda i: (0, 0)),
        ],
        out_specs=pl.BlockSpec((TAIL_ROWS, NUM_CLASSES), lambda i: (i, 0)),
        out_shape=jax.ShapeDtypeStruct((B, NUM_CLASSES), jnp.float32),
    )(text_rep, labels, comb, model_prob,
      W_enc, b_enc.reshape(1, D), W1, b1.reshape(1, 2 * D), W2,
      b2.reshape(1, 1))


def kernel(x, x_mask, x_idx, keys_store, store_vals, neighbor_feats,
           W_enc, b_enc, W_cls, b_cls, W1, b1, W2, b2):
    q = keys_store[x_idx]
    scores, gm_t = _scores(q, keys_store, x_idx)
    gm1 = gm_t.transpose(2, 0, 1).reshape(B, G)

    labels, comb = _select(gm1, scores.reshape(B * G, 128),
                           keys_store, neighbor_feats, store_vals)

    text_rep, model_prob = _encode(x, x_mask, W_enc, b_enc, W_cls, b_cls)

    return _tail(text_rep, labels, comb, model_prob,
                 W_enc, b_enc, W1, b1, W2, b2)
